# Initial kernel scaffold; baseline (speedup 1.0000x reference)
import jax
import jax.numpy as jnp
from jax.experimental import pallas as pl


def kernel(x, c1w, b1g, b1b, l1_0_c1w, l1_0_b1g, l1_0_b1b, l1_0_c2w, l1_0_b2g, l1_0_b2b, l1_1_c1w, l1_1_b1g, l1_1_b1b, l1_1_c2w, l1_1_b2g, l1_1_b2b, l2_0_c1w, l2_0_b1g, l2_0_b1b, l2_0_c2w, l2_0_b2g, l2_0_b2b, l2_0_dw, l2_0_dbg, l2_0_dbb, l2_1_c1w, l2_1_b1g, l2_1_b1b, l2_1_c2w, l2_1_b2g, l2_1_b2b, l3_0_c1w, l3_0_b1g, l3_0_b1b, l3_0_c2w, l3_0_b2g, l3_0_b2b, l3_0_dw, l3_0_dbg, l3_0_dbb, l3_1_c1w, l3_1_b1g, l3_1_b1b, l3_1_c2w, l3_1_b2g, l3_1_b2b, l4_0_c1w, l4_0_b1g, l4_0_b1b, l4_0_c2w, l4_0_b2g, l4_0_b2b, l4_0_dw, l4_0_dbg, l4_0_dbb, l4_1_c1w, l4_1_b1g, l4_1_b1b, l4_1_c2w, l4_1_b2g, l4_1_b2b, fcw, fcb):
    raise NotImplementedError("write your pallas kernel here")


# speedup vs baseline: 3.5275x; 3.5275x over previous
import functools

import jax
import jax.numpy as jnp
from jax.experimental import pallas as pl
from jax.experimental.pallas import tpu as pltpu

_EPS = 1e-5


def _bn_coeffs(sum_ref, ssq_ref, g_ref, b_ref, count):
    s = jnp.sum(sum_ref[...], axis=0)
    q = jnp.sum(ssq_ref[...], axis=0)
    inv = 1.0 / count
    mean = s * inv
    var = jnp.maximum(q * inv - mean * mean, 0.0)
    scale = g_ref[...] * jax.lax.rsqrt(var + _EPS)
    shift = b_ref[...] - mean * scale
    return scale, shift


def _conv_body(*refs, NB, H, W, C, OC, has_bn, count_in):
    it = iter(refs)
    x_ref = next(it)
    if has_bn:
        psum, pssq, g_ref, b_ref = next(it), next(it), next(it), next(it)
    w_ref = next(it)
    o_ref, os_ref, oq_ref = next(it), next(it), next(it)
    pr = next(it)

    M = NB * H * W
    xb = x_ref[...]
    if has_bn:
        scale, shift = _bn_coeffs(psum, pssq, g_ref, b_ref, count_in)
        a = jnp.maximum(xb.astype(jnp.float32) * scale + shift, 0.0)
        a = a.astype(jnp.bfloat16)
    else:
        a = xb

    pr[...] = jnp.zeros_like(pr)
    pr[:, pl.ds(1, H), pl.ds(1, W), :] = a.reshape(NB, H, W, C)

    acc = None
    for dy in range(3):
        for dx in range(3):
            sl = pr[:, pl.ds(dy, H), pl.ds(dx, W), :]
            at = sl.reshape(M, C)
            d = jnp.dot(at, w_ref[dy * 3 + dx],
                        preferred_element_type=jnp.float32)
            acc = d if acc is None else acc + d

    o_ref[...] = acc.astype(jnp.bfloat16)
    os_ref[...] = jnp.sum(acc, axis=0, keepdims=True)[None]
    oq_ref[...] = jnp.sum(acc * acc, axis=0, keepdims=True)[None]


def _conv_s2_body(p00, p01, p10, p11, w_ref, wd_ref, o_ref, os_ref, oq_ref,
                  od_ref, ods_ref, odq_ref, *, NB, OH, C):
    ph = (p00, p01, p10, p11)
    M = NB * OH * OH
    acc = None
    for dy in range(3):
        for dx in range(3):
            ref = ph[(dy % 2) * 2 + (dx % 2)]
            sl = ref[:, pl.ds(dy // 2, OH), pl.ds(dx // 2, OH), :]
            at = sl.reshape(M, C)
            d = jnp.dot(at, w_ref[dy * 3 + dx],
                        preferred_element_type=jnp.float32)
            acc = d if acc is None else acc + d
    o_ref[...] = acc.astype(jnp.bfloat16)
    os_ref[...] = jnp.sum(acc, axis=0, keepdims=True)[None]
    oq_ref[...] = jnp.sum(acc * acc, axis=0, keepdims=True)[None]

    ad = p11[:, pl.ds(0, OH), pl.ds(0, OH), :]
    accd = jnp.dot(ad.reshape(M, C), wd_ref[0],
                   preferred_element_type=jnp.float32)
    od_ref[...] = accd.astype(jnp.bfloat16)
    ods_ref[...] = jnp.sum(accd, axis=0, keepdims=True)[None]
    odq_ref[...] = jnp.sum(accd * accd, axis=0, keepdims=True)[None]


def _stat_specs(G, OC):
    return [
        pl.BlockSpec((1, 1, OC), lambda i: (i, 0, 0)),
        pl.BlockSpec((1, 1, OC), lambda i: (i, 0, 0)),
    ]


def _stat_shapes(G, OC):
    return [
        jax.ShapeDtypeStruct((G, 1, OC), jnp.float32),
        jax.ShapeDtypeStruct((G, 1, OC), jnp.float32),
    ]


def _conv(x2, w9, *, NB, H, W, C, OC, stats_in=None):
    N = 64
    G = N // NB
    M = NB * H * W
    has_bn = stats_in is not None

    in_specs = [pl.BlockSpec((M, C), lambda i: (i, 0))]
    args = [x2]
    if has_bn:
        s_in, q_in, g_in, b_in = stats_in
        gp = s_in.shape[0]
        in_specs += [
            pl.BlockSpec((gp, 1, C), lambda i: (0, 0, 0)),
            pl.BlockSpec((gp, 1, C), lambda i: (0, 0, 0)),
            pl.BlockSpec((1, C), lambda i: (0, 0)),
            pl.BlockSpec((1, C), lambda i: (0, 0)),
        ]
        args += [s_in, q_in, g_in.reshape(1, C).astype(jnp.float32),
                 b_in.reshape(1, C).astype(jnp.float32)]
    in_specs.append(pl.BlockSpec((9, C, OC), lambda i: (0, 0, 0)))
    args.append(w9)

    body = functools.partial(_conv_body, NB=NB, H=H, W=W, C=C, OC=OC,
                             has_bn=has_bn, count_in=float(N * H * W))

    return pl.pallas_call(
        body,
        out_shape=[jax.ShapeDtypeStruct((N * H * W, OC), jnp.bfloat16)]
        + _stat_shapes(G, OC),
        grid_spec=pltpu.PrefetchScalarGridSpec(
            num_scalar_prefetch=0,
            grid=(G,),
            in_specs=in_specs,
            out_specs=[pl.BlockSpec((M, OC), lambda i: (i, 0))]
            + _stat_specs(G, OC),
            scratch_shapes=[
                pltpu.VMEM((NB, H + 2, W + 2, C), jnp.bfloat16)]),
        compiler_params=pltpu.CompilerParams(
            dimension_semantics=("parallel",)),
    )(*args)


def _phases(x2, H, C, pad):
    x4 = x2.reshape(64, H, H, C)
    if pad:
        x4 = jnp.pad(x4, ((0, 0), (1, 1), (1, 1), (0, 0)))
    return [x4[:, r::2, s::2, :] for r in (0, 1) for s in (0, 1)]


def _conv_s2(ph, w9, wd, *, NB, H, C, OC):
    N = 64
    G = N // NB
    OH = H // 2
    PH = (H + 2) // 2
    M = NB * OH * OH
    OCD = wd.shape[2]

    ph_spec = pl.BlockSpec((NB, PH, PH, C), lambda i: (i, 0, 0, 0))
    body = functools.partial(_conv_s2_body, NB=NB, OH=OH, C=C)
    return pl.pallas_call(
        body,
        out_shape=[jax.ShapeDtypeStruct((N * OH * OH, OC), jnp.bfloat16)]
        + _stat_shapes(G, OC)
        + [jax.ShapeDtypeStruct((N * OH * OH, OCD), jnp.bfloat16)]
        + _stat_shapes(G, OCD),
        grid_spec=pltpu.PrefetchScalarGridSpec(
            num_scalar_prefetch=0,
            grid=(G,),
            in_specs=[ph_spec, ph_spec, ph_spec, ph_spec,
                      pl.BlockSpec((9, C, OC), lambda i: (0, 0, 0)),
                      pl.BlockSpec((1, C, OCD), lambda i: (0, 0, 0))],
            out_specs=[pl.BlockSpec((M, OC), lambda i: (i, 0))]
            + _stat_specs(G, OC)
            + [pl.BlockSpec((M, OCD), lambda i: (i, 0))]
            + _stat_specs(G, OCD)),
        compiler_params=pltpu.CompilerParams(
            dimension_semantics=("parallel",)),
    )(*ph, w9, wd)


def _residual_body(y_ref, ys_ref, yq_ref, yg_ref, yb_ref, r_ref, *rest,
                   count, count_d, has_dstats):
    if has_dstats:
        rs_ref, rq_ref, rg_ref, rb_ref, o_ref = rest
    else:
        (o_ref,) = rest
    scale, shift = _bn_coeffs(ys_ref, yq_ref, yg_ref, yb_ref, count)
    y = y_ref[...].astype(jnp.float32) * scale + shift
    if has_dstats:
        ds, dh = _bn_coeffs(rs_ref, rq_ref, rg_ref, rb_ref, count_d)
        r = r_ref[...].astype(jnp.float32) * ds + dh
    else:
        r = r_ref[...].astype(jnp.float32)
    o_ref[...] = jnp.maximum(y + r, 0.0).astype(jnp.bfloat16)


def _residual(y2, stats2, res2, statsd, *, rows, C, G, count, count_d):
    TR = rows // G
    s2, q2, g2, b2 = stats2
    gp = s2.shape[0]
    row_spec = pl.BlockSpec((TR, C), lambda i: (i, 0))
    st_spec = pl.BlockSpec((gp, 1, C), lambda i: (0, 0, 0))
    vec_spec = pl.BlockSpec((1, C), lambda i: (0, 0))
    in_specs = [row_spec, st_spec, st_spec, vec_spec, vec_spec, row_spec]
    args = [y2, s2, q2, g2.reshape(1, C).astype(jnp.float32),
            b2.reshape(1, C).astype(jnp.float32), res2]
    if statsd is not None:
        sd, qd, gd, bd = statsd
        gpd = sd.shape[0]
        std_spec = pl.BlockSpec((gpd, 1, C), lambda i: (0, 0, 0))
        in_specs += [std_spec, std_spec, vec_spec, vec_spec]
        args += [sd, qd, gd.reshape(1, C).astype(jnp.float32),
                 bd.reshape(1, C).astype(jnp.float32)]
    body = functools.partial(_residual_body, count=count, count_d=count_d,
                             has_dstats=statsd is not None)
    return pl.pallas_call(
        body,
        out_shape=jax.ShapeDtypeStruct((rows, C), jnp.bfloat16),
        grid_spec=pltpu.PrefetchScalarGridSpec(
            num_scalar_prefetch=0,
            grid=(G,),
            in_specs=in_specs,
            out_specs=row_spec),
        compiler_params=pltpu.CompilerParams(
            dimension_semantics=("parallel",)),
    )(*args)


def _stem_body(x_ref, w_ref, o_ref, os_ref, oq_ref, acc_ref):
    for a in range(4):
        for b in range(4):
            sl = x_ref[0, 0, pl.ds(a, 28), pl.ds(b, 112), :]
            at = sl.reshape(28 * 112, 12)
            d = jnp.dot(at, w_ref[a * 4 + b],
                        preferred_element_type=jnp.float32)
            if a == 0 and b == 0:
                acc_ref[...] = d
            else:
                acc_ref[...] += d
    acc = acc_ref[...]
    o_ref[...] = acc.astype(jnp.bfloat16)
    os_ref[...] = jnp.sum(acc, axis=0, keepdims=True)[None]
    oq_ref[...] = jnp.sum(acc * acc, axis=0, keepdims=True)[None]


def _stem(xs, w16):
    M = 28 * 112
    return pl.pallas_call(
        _stem_body,
        out_shape=[
            jax.ShapeDtypeStruct((64 * 112 * 112, 64), jnp.bfloat16),
            jax.ShapeDtypeStruct((256, 1, 64), jnp.float32),
            jax.ShapeDtypeStruct((256, 1, 64), jnp.float32),
        ],
        grid_spec=pltpu.PrefetchScalarGridSpec(
            num_scalar_prefetch=0,
            grid=(64, 4),
            in_specs=[
                pl.BlockSpec((1, 1, 31, 115, 12),
                             lambda i, s: (i, s, 0, 0, 0)),
                pl.BlockSpec((16, 12, 64), lambda i, s: (0, 0, 0)),
            ],
            out_specs=[
                pl.BlockSpec((M, 64), lambda i, s: (i * 4 + s, 0)),
                pl.BlockSpec((1, 1, 64), lambda i, s: (i * 4 + s, 0, 0)),
                pl.BlockSpec((1, 1, 64), lambda i, s: (i * 4 + s, 0, 0)),
            ],
            scratch_shapes=[pltpu.VMEM((M, 64), jnp.float32)]),
        compiler_params=pltpu.CompilerParams(
            dimension_semantics=("parallel", "parallel")),
    )(xs, w16)


def _pool_body(q00, q01, q10, q11, psum, pssq, g_ref, b_ref, o_ref,
               s01, s10, s11, *, count):
    scale, shift = _bn_coeffs(psum, pssq, g_ref, b_ref, count)

    def bn(qref):
        v = qref[0].astype(jnp.float32)
        return jnp.maximum(v * scale + shift, 0.0)

    s01[...] = jnp.full_like(s01, -jnp.inf)
    s01[:, pl.ds(1, 56), :] = bn(q01)
    s10[...] = jnp.full_like(s10, -jnp.inf)
    s10[pl.ds(1, 56), :, :] = bn(q10)
    s11[...] = jnp.full_like(s11, -jnp.inf)
    s11[pl.ds(1, 56), pl.ds(1, 56), :] = bn(q11)

    m = bn(q00)
    m = jnp.maximum(m, s01[:, pl.ds(0, 56), :])
    m = jnp.maximum(m, s01[:, pl.ds(1, 56), :])
    m = jnp.maximum(m, s10[pl.ds(0, 56), :, :])
    m = jnp.maximum(m, s10[pl.ds(1, 56), :, :])
    m = jnp.maximum(m, s11[pl.ds(0, 56), pl.ds(0, 56), :])
    m = jnp.maximum(m, s11[pl.ds(0, 56), pl.ds(1, 56), :])
    m = jnp.maximum(m, s11[pl.ds(1, 56), pl.ds(0, 56), :])
    m = jnp.maximum(m, s11[pl.ds(1, 56), pl.ds(1, 56), :])
    o_ref[...] = m.reshape(56 * 56, 64).astype(jnp.bfloat16)


def _pool(y0, stats0):
    s0, q0, g0, b0 = stats0
    qs = _phases(y0, 112, 64, pad=False)
    body = functools.partial(_pool_body, count=float(64 * 112 * 112))
    q_spec = pl.BlockSpec((1, 56, 56, 64), lambda i: (i, 0, 0, 0))
    return pl.pallas_call(
        body,
        out_shape=jax.ShapeDtypeStruct((64 * 56 * 56, 64), jnp.bfloat16),
        grid_spec=pltpu.PrefetchScalarGridSpec(
            num_scalar_prefetch=0,
            grid=(64,),
            in_specs=[
                q_spec, q_spec, q_spec, q_spec,
                pl.BlockSpec((256, 1, 64), lambda i: (0, 0, 0)),
                pl.BlockSpec((256, 1, 64), lambda i: (0, 0, 0)),
                pl.BlockSpec((1, 64), lambda i: (0, 0)),
                pl.BlockSpec((1, 64), lambda i: (0, 0)),
            ],
            out_specs=pl.BlockSpec((56 * 56, 64), lambda i: (i, 0)),
            scratch_shapes=[
                pltpu.VMEM((56, 57, 64), jnp.float32),
                pltpu.VMEM((57, 56, 64), jnp.float32),
                pltpu.VMEM((57, 57, 64), jnp.float32),
            ]),
        compiler_params=pltpu.CompilerParams(
            dimension_semantics=("parallel",)),
    )(*qs, s0, q0, g0.reshape(1, 64).astype(jnp.float32),
      b0.reshape(1, 64).astype(jnp.float32))


def _head_body(x_ref, w_ref, b_ref, o_ref):
    xm = jnp.mean(x_ref[...].astype(jnp.float32), axis=1)
    o_ref[...] = (jnp.dot(xm, w_ref[...],
                          preferred_element_type=jnp.float32) + b_ref[...])


def _head(x4, fcw, fcb):
    x3 = x4.reshape(64, 49, 512)
    wT = jnp.transpose(fcw).astype(jnp.float32)
    b2 = fcb.reshape(1, 1000).astype(jnp.float32)
    return pl.pallas_call(
        _head_body,
        out_shape=jax.ShapeDtypeStruct((64, 1000), jnp.float32),
        grid_spec=pltpu.PrefetchScalarGridSpec(
            num_scalar_prefetch=0,
            grid=(1,),
            in_specs=[
                pl.BlockSpec((64, 49, 512), lambda i: (0, 0, 0)),
                pl.BlockSpec((512, 1000), lambda i: (0, 0)),
                pl.BlockSpec((1, 1000), lambda i: (0, 0)),
            ],
            out_specs=pl.BlockSpec((64, 1000), lambda i: (0, 0))),
        compiler_params=pltpu.CompilerParams(
            dimension_semantics=("arbitrary",)),
    )(x3, wT, b2)


def _w9(w):
    OC, IC, KH, KW = w.shape
    return jnp.transpose(w, (2, 3, 1, 0)).reshape(KH * KW, IC, OC).astype(
        jnp.bfloat16)


def _basic_block(x2, p, *, H, C, OC, stride, NB_in, NB_out):
    OH = H // stride
    count_o = float(64 * OH * OH)
    wd = p.get('down_w')
    if stride == 2:
        ph = _phases(x2, H, C, pad=True)
        y1, s1, q1, *down = _conv_s2(
            ph, _w9(p['conv1_w']), _w9(wd), NB=NB_in, H=H, C=C, OC=OC)
    else:
        y1, s1, q1 = _conv(
            x2, _w9(p['conv1_w']), NB=NB_in, H=H, W=H, C=C, OC=OC)
        down = []
    stats1 = (s1, q1, p['bn1_g'], p['bn1_b'])
    y2, s2, q2 = _conv(
        y1, _w9(p['conv2_w']), NB=NB_out, H=OH, W=OH, C=OC, OC=OC,
        stats_in=stats1)
    stats2 = (s2, q2, p['bn2_g'], p['bn2_b'])
    if wd is not None:
        yd, sd, qd = down
        statsd = (sd, qd, p['down_bn_g'], p['down_bn_b'])
        res = yd
    else:
        statsd = None
        res = x2
    G = 64 // NB_out
    return _residual(y2, stats2, res, statsd,
                     rows=64 * OH * OH, C=OC, G=G,
                     count=count_o, count_d=count_o)


def kernel(x, c1w, b1g, b1b, l1_0_c1w, l1_0_b1g, l1_0_b1b, l1_0_c2w, l1_0_b2g, l1_0_b2b, l1_1_c1w, l1_1_b1g, l1_1_b1b, l1_1_c2w, l1_1_b2g, l1_1_b2b, l2_0_c1w, l2_0_b1g, l2_0_b1b, l2_0_c2w, l2_0_b2g, l2_0_b2b, l2_0_dw, l2_0_dbg, l2_0_dbb, l2_1_c1w, l2_1_b1g, l2_1_b1b, l2_1_c2w, l2_1_b2g, l2_1_b2b, l3_0_c1w, l3_0_b1g, l3_0_b1b, l3_0_c2w, l3_0_b2g, l3_0_b2b, l3_0_dw, l3_0_dbg, l3_0_dbb, l3_1_c1w, l3_1_b1g, l3_1_b1b, l3_1_c2w, l3_1_b2g, l3_1_b2b, l4_0_c1w, l4_0_b1g, l4_0_b1b, l4_0_c2w, l4_0_b2g, l4_0_b2b, l4_0_dw, l4_0_dbg, l4_0_dbb, l4_1_c1w, l4_1_b1g, l4_1_b1b, l4_1_c2w, l4_1_b2g, l4_1_b2b, fcw, fcb):
    xp = jnp.pad(x, ((0, 0), (0, 0), (3, 3), (3, 3)))
    xs = xp.reshape(64, 3, 115, 2, 115, 2).transpose(
        0, 2, 4, 3, 5, 1).reshape(64, 115, 115, 12).astype(jnp.bfloat16)
    xs = jnp.stack([xs[:, 28 * s:28 * s + 31] for s in range(4)], axis=1)
    w8 = jnp.pad(c1w, ((0, 0), (0, 0), (0, 1), (0, 1)))
    w16 = w8.reshape(64, 3, 4, 2, 4, 2).transpose(
        2, 4, 3, 5, 1, 0).reshape(16, 12, 64).astype(jnp.bfloat16)
    y0, s0, q0 = _stem(xs, w16)
    x1 = _pool(y0, (s0, q0, b1g, b1b))

    p_l1_0 = {'conv1_w': l1_0_c1w, 'bn1_g': l1_0_b1g, 'bn1_b': l1_0_b1b,
              'conv2_w': l1_0_c2w, 'bn2_g': l1_0_b2g, 'bn2_b': l1_0_b2b}
    p_l1_1 = {'conv1_w': l1_1_c1w, 'bn1_g': l1_1_b1g, 'bn1_b': l1_1_b1b,
              'conv2_w': l1_1_c2w, 'bn2_g': l1_1_b2g, 'bn2_b': l1_1_b2b}
    p_l2_0 = {'conv1_w': l2_0_c1w, 'bn1_g': l2_0_b1g, 'bn1_b': l2_0_b1b,
              'conv2_w': l2_0_c2w, 'bn2_g': l2_0_b2g, 'bn2_b': l2_0_b2b,
              'down_w': l2_0_dw, 'down_bn_g': l2_0_dbg, 'down_bn_b': l2_0_dbb}
    p_l2_1 = {'conv1_w': l2_1_c1w, 'bn1_g': l2_1_b1g, 'bn1_b': l2_1_b1b,
              'conv2_w': l2_1_c2w, 'bn2_g': l2_1_b2g, 'bn2_b': l2_1_b2b}
    p_l3_0 = {'conv1_w': l3_0_c1w, 'bn1_g': l3_0_b1g, 'bn1_b': l3_0_b1b,
              'conv2_w': l3_0_c2w, 'bn2_g': l3_0_b2g, 'bn2_b': l3_0_b2b,
              'down_w': l3_0_dw, 'down_bn_g': l3_0_dbg, 'down_bn_b': l3_0_dbb}
    p_l3_1 = {'conv1_w': l3_1_c1w, 'bn1_g': l3_1_b1g, 'bn1_b': l3_1_b1b,
              'conv2_w': l3_1_c2w, 'bn2_g': l3_1_b2g, 'bn2_b': l3_1_b2b}
    p_l4_0 = {'conv1_w': l4_0_c1w, 'bn1_g': l4_0_b1g, 'bn1_b': l4_0_b1b,
              'conv2_w': l4_0_c2w, 'bn2_g': l4_0_b2g, 'bn2_b': l4_0_b2b,
              'down_w': l4_0_dw, 'down_bn_g': l4_0_dbg, 'down_bn_b': l4_0_dbb}
    p_l4_1 = {'conv1_w': l4_1_c1w, 'bn1_g': l4_1_b1g, 'bn1_b': l4_1_b1b,
              'conv2_w': l4_1_c2w, 'bn2_g': l4_1_b2g, 'bn2_b': l4_1_b2b}

    x = _basic_block(x1, p_l1_0, H=56, C=64, OC=64, stride=1,
                     NB_in=1, NB_out=1)
    x = _basic_block(x, p_l1_1, H=56, C=64, OC=64, stride=1,
                     NB_in=1, NB_out=1)
    x = _basic_block(x, p_l2_0, H=56, C=64, OC=128, stride=2,
                     NB_in=4, NB_out=4)
    x = _basic_block(x, p_l2_1, H=28, C=128, OC=128, stride=1,
                     NB_in=4, NB_out=4)
    x = _basic_block(x, p_l3_0, H=28, C=128, OC=256, stride=2,
                     NB_in=8, NB_out=8)
    x = _basic_block(x, p_l3_1, H=14, C=256, OC=256, stride=1,
                     NB_in=8, NB_out=8)
    x = _basic_block(x, p_l4_0, H=14, C=256, OC=512, stride=2,
                     NB_in=16, NB_out=16)
    x = _basic_block(x, p_l4_1, H=7, C=512, OC=512, stride=1,
                     NB_in=16, NB_out=16)

    return _head(x, fcw, fcb)


# speedup vs baseline: 5.5740x; 1.5802x over previous
import functools

import jax
import jax.numpy as jnp
from jax.experimental import pallas as pl
from jax.experimental.pallas import tpu as pltpu

_EPS = 1e-5


def _bn_coeffs(sum_ref, ssq_ref, g_ref, b_ref, count):
    s = jnp.sum(sum_ref[...], axis=0)
    q = jnp.sum(ssq_ref[...], axis=0)
    inv = 1.0 / count
    mean = s * inv
    var = jnp.maximum(q * inv - mean * mean, 0.0)
    scale = g_ref[...] * jax.lax.rsqrt(var + _EPS)
    shift = b_ref[...] - mean * scale
    return scale, shift


def _conv_body(*refs, NB, H, W, C, OC, has_bn, count_in):
    it = iter(refs)
    x_ref = next(it)
    if has_bn:
        psum, pssq, g_ref, b_ref = next(it), next(it), next(it), next(it)
    w_ref = next(it)
    o_ref, os_ref, oq_ref = next(it), next(it), next(it)
    pr = next(it)

    M = NB * H * W
    xb = x_ref[...]
    if has_bn:
        scale, shift = _bn_coeffs(psum, pssq, g_ref, b_ref, count_in)
        a = jnp.maximum(xb.astype(jnp.float32) * scale + shift, 0.0)
        a = a.astype(jnp.bfloat16)
    else:
        a = xb

    pr[...] = jnp.zeros_like(pr)
    pr[:, pl.ds(1, H), pl.ds(1, W), :] = a.reshape(NB, H, W, C)

    acc = None
    for dy in range(3):
        for dx in range(3):
            sl = pr[:, pl.ds(dy, H), pl.ds(dx, W), :]
            at = sl.reshape(M, C)
            d = jnp.dot(at, w_ref[dy * 3 + dx],
                        preferred_element_type=jnp.float32)
            acc = d if acc is None else acc + d

    o_ref[...] = acc.astype(jnp.bfloat16)
    os_ref[...] = jnp.sum(acc, axis=0, keepdims=True)[None]
    oq_ref[...] = jnp.sum(acc * acc, axis=0, keepdims=True)[None]


def _conv_s2_body(p00, p01, p10, p11, w_ref, wd_ref, o_ref, os_ref, oq_ref,
                  od_ref, ods_ref, odq_ref, *, NB, OH, C):
    ph = (p00, p01, p10, p11)
    M = NB * OH * OH
    acc = None
    for dy in range(3):
        for dx in range(3):
            ref = ph[(dy % 2) * 2 + (dx % 2)]
            sl = ref[:, pl.ds(dy // 2, OH), pl.ds(dx // 2, OH), :]
            at = sl.reshape(M, C)
            d = jnp.dot(at, w_ref[dy * 3 + dx],
                        preferred_element_type=jnp.float32)
            acc = d if acc is None else acc + d
    o_ref[...] = acc.astype(jnp.bfloat16)
    os_ref[...] = jnp.sum(acc, axis=0, keepdims=True)[None]
    oq_ref[...] = jnp.sum(acc * acc, axis=0, keepdims=True)[None]

    ad = p11[:, pl.ds(0, OH), pl.ds(0, OH), :]
    accd = jnp.dot(ad.reshape(M, C), wd_ref[0],
                   preferred_element_type=jnp.float32)
    od_ref[...] = accd.astype(jnp.bfloat16)
    ods_ref[...] = jnp.sum(accd, axis=0, keepdims=True)[None]
    odq_ref[...] = jnp.sum(accd * accd, axis=0, keepdims=True)[None]


def _stat_specs(G, OC):
    return [
        pl.BlockSpec((1, 1, OC), lambda i: (i, 0, 0)),
        pl.BlockSpec((1, 1, OC), lambda i: (i, 0, 0)),
    ]


def _stat_shapes(G, OC):
    return [
        jax.ShapeDtypeStruct((G, 1, OC), jnp.float32),
        jax.ShapeDtypeStruct((G, 1, OC), jnp.float32),
    ]


def _conv(x2, w9, *, NB, H, W, C, OC, stats_in=None):
    N = 64
    G = N // NB
    M = NB * H * W
    has_bn = stats_in is not None

    in_specs = [pl.BlockSpec((M, C), lambda i: (i, 0))]
    args = [x2]
    if has_bn:
        s_in, q_in, g_in, b_in = stats_in
        gp = s_in.shape[0]
        in_specs += [
            pl.BlockSpec((gp, 1, C), lambda i: (0, 0, 0)),
            pl.BlockSpec((gp, 1, C), lambda i: (0, 0, 0)),
            pl.BlockSpec((1, C), lambda i: (0, 0)),
            pl.BlockSpec((1, C), lambda i: (0, 0)),
        ]
        args += [s_in, q_in, g_in.reshape(1, C).astype(jnp.float32),
                 b_in.reshape(1, C).astype(jnp.float32)]
    in_specs.append(pl.BlockSpec((9, C, OC), lambda i: (0, 0, 0)))
    args.append(w9)

    body = functools.partial(_conv_body, NB=NB, H=H, W=W, C=C, OC=OC,
                             has_bn=has_bn, count_in=float(N * H * W))

    return pl.pallas_call(
        body,
        out_shape=[jax.ShapeDtypeStruct((N * H * W, OC), jnp.bfloat16)]
        + _stat_shapes(G, OC),
        grid_spec=pltpu.PrefetchScalarGridSpec(
            num_scalar_prefetch=0,
            grid=(G,),
            in_specs=in_specs,
            out_specs=[pl.BlockSpec((M, OC), lambda i: (i, 0))]
            + _stat_specs(G, OC),
            scratch_shapes=[
                pltpu.VMEM((NB, H + 2, W + 2, C), jnp.bfloat16)]),
        compiler_params=pltpu.CompilerParams(
            dimension_semantics=("parallel",)),
    )(*args)


def _phases(x2, H, C, pad):
    x4 = x2.reshape(64, H, H, C)
    if pad:
        x4 = jnp.pad(x4, ((0, 0), (1, 1), (1, 1), (0, 0)))
    return [x4[:, r::2, s::2, :] for r in (0, 1) for s in (0, 1)]


def _conv_s2(ph, w9, wd, *, NB, H, C, OC):
    N = 64
    G = N // NB
    OH = H // 2
    PH = (H + 2) // 2
    M = NB * OH * OH
    OCD = wd.shape[2]

    ph_spec = pl.BlockSpec((NB, PH, PH, C), lambda i: (i, 0, 0, 0))
    body = functools.partial(_conv_s2_body, NB=NB, OH=OH, C=C)
    return pl.pallas_call(
        body,
        out_shape=[jax.ShapeDtypeStruct((N * OH * OH, OC), jnp.bfloat16)]
        + _stat_shapes(G, OC)
        + [jax.ShapeDtypeStruct((N * OH * OH, OCD), jnp.bfloat16)]
        + _stat_shapes(G, OCD),
        grid_spec=pltpu.PrefetchScalarGridSpec(
            num_scalar_prefetch=0,
            grid=(G,),
            in_specs=[ph_spec, ph_spec, ph_spec, ph_spec,
                      pl.BlockSpec((9, C, OC), lambda i: (0, 0, 0)),
                      pl.BlockSpec((1, C, OCD), lambda i: (0, 0, 0))],
            out_specs=[pl.BlockSpec((M, OC), lambda i: (i, 0))]
            + _stat_specs(G, OC)
            + [pl.BlockSpec((M, OCD), lambda i: (i, 0))]
            + _stat_specs(G, OCD)),
        compiler_params=pltpu.CompilerParams(
            dimension_semantics=("parallel",)),
    )(*ph, w9, wd)


def _residual_body(y_ref, ys_ref, yq_ref, yg_ref, yb_ref, r_ref, *rest,
                   count, count_d, has_dstats):
    if has_dstats:
        rs_ref, rq_ref, rg_ref, rb_ref, o_ref = rest
    else:
        (o_ref,) = rest
    scale, shift = _bn_coeffs(ys_ref, yq_ref, yg_ref, yb_ref, count)
    y = y_ref[...].astype(jnp.float32) * scale + shift
    if has_dstats:
        ds, dh = _bn_coeffs(rs_ref, rq_ref, rg_ref, rb_ref, count_d)
        r = r_ref[...].astype(jnp.float32) * ds + dh
    else:
        r = r_ref[...].astype(jnp.float32)
    o_ref[...] = jnp.maximum(y + r, 0.0).astype(jnp.bfloat16)


def _residual(y2, stats2, res2, statsd, *, rows, C, G, count, count_d):
    TR = rows // G
    s2, q2, g2, b2 = stats2
    gp = s2.shape[0]
    row_spec = pl.BlockSpec((TR, C), lambda i: (i, 0))
    st_spec = pl.BlockSpec((gp, 1, C), lambda i: (0, 0, 0))
    vec_spec = pl.BlockSpec((1, C), lambda i: (0, 0))
    in_specs = [row_spec, st_spec, st_spec, vec_spec, vec_spec, row_spec]
    args = [y2, s2, q2, g2.reshape(1, C).astype(jnp.float32),
            b2.reshape(1, C).astype(jnp.float32), res2]
    if statsd is not None:
        sd, qd, gd, bd = statsd
        gpd = sd.shape[0]
        std_spec = pl.BlockSpec((gpd, 1, C), lambda i: (0, 0, 0))
        in_specs += [std_spec, std_spec, vec_spec, vec_spec]
        args += [sd, qd, gd.reshape(1, C).astype(jnp.float32),
                 bd.reshape(1, C).astype(jnp.float32)]
    body = functools.partial(_residual_body, count=count, count_d=count_d,
                             has_dstats=statsd is not None)
    return pl.pallas_call(
        body,
        out_shape=jax.ShapeDtypeStruct((rows, C), jnp.bfloat16),
        grid_spec=pltpu.PrefetchScalarGridSpec(
            num_scalar_prefetch=0,
            grid=(G,),
            in_specs=in_specs,
            out_specs=row_spec),
        compiler_params=pltpu.CompilerParams(
            dimension_semantics=("parallel",)),
    )(*args)


def _stem_body(q00, q01, q10, q11, w_ref, o_ref, os_ref, oq_ref, acc_ref):
    qs = (q00, q01, q10, q11)
    ssum = None
    for r in (0, 1):
        for c in (0, 1):
            for a in range(4):
                for b in range(4):
                    u, v = r + a, c + b
                    ref = qs[(u % 2) * 2 + (v % 2)]
                    sl = ref[0, pl.ds(u // 2, 56), pl.ds(v // 2, 56), :]
                    at = sl.reshape(56 * 56, 12)
                    d = jnp.dot(at, w_ref[a * 4 + b],
                                preferred_element_type=jnp.float32)
                    if a == 0 and b == 0:
                        acc_ref[...] = d
                    else:
                        acc_ref[...] += d
            acc = acc_ref[...]
            o_ref[0, r * 2 + c] = acc.reshape(56, 56, 64).astype(jnp.bfloat16)
            s1 = jnp.sum(acc, axis=0, keepdims=True)
            s2 = jnp.sum(acc * acc, axis=0, keepdims=True)
            ssum = (s1, s2) if ssum is None else (ssum[0] + s1, ssum[1] + s2)
    os_ref[...] = ssum[0][None]
    oq_ref[...] = ssum[1][None]


def _stem(xq, w16):
    q_spec = pl.BlockSpec((1, 58, 58, 12), lambda i: (i, 0, 0, 0))
    return pl.pallas_call(
        _stem_body,
        out_shape=[
            jax.ShapeDtypeStruct((64, 4, 56, 56, 64), jnp.bfloat16),
            jax.ShapeDtypeStruct((64, 1, 64), jnp.float32),
            jax.ShapeDtypeStruct((64, 1, 64), jnp.float32),
        ],
        grid_spec=pltpu.PrefetchScalarGridSpec(
            num_scalar_prefetch=0,
            grid=(64,),
            in_specs=[q_spec, q_spec, q_spec, q_spec,
                      pl.BlockSpec((16, 12, 64), lambda i: (0, 0, 0))],
            out_specs=[
                pl.BlockSpec((1, 4, 56, 56, 64), lambda i: (i, 0, 0, 0, 0)),
                pl.BlockSpec((1, 1, 64), lambda i: (i, 0, 0)),
                pl.BlockSpec((1, 1, 64), lambda i: (i, 0, 0)),
            ],
            scratch_shapes=[pltpu.VMEM((56 * 56, 64), jnp.float32)]),
        compiler_params=pltpu.CompilerParams(
            dimension_semantics=("parallel",)),
    )(*xq, w16)


def _pool_body(q00, q01, q10, q11, psum, pssq, g_ref, b_ref, o_ref,
               s01, s10, s11, *, count):
    scale, shift = _bn_coeffs(psum, pssq, g_ref, b_ref, count)

    def bn(qref):
        v = qref[0, 0].astype(jnp.float32)
        return jnp.maximum(v * scale + shift, 0.0)

    s01[...] = jnp.full_like(s01, -jnp.inf)
    s01[:, pl.ds(1, 56), :] = bn(q01)
    s10[...] = jnp.full_like(s10, -jnp.inf)
    s10[pl.ds(1, 56), :, :] = bn(q10)
    s11[...] = jnp.full_like(s11, -jnp.inf)
    s11[pl.ds(1, 56), pl.ds(1, 56), :] = bn(q11)

    m = bn(q00)
    m = jnp.maximum(m, s01[:, pl.ds(0, 56), :])
    m = jnp.maximum(m, s01[:, pl.ds(1, 56), :])
    m = jnp.maximum(m, s10[pl.ds(0, 56), :, :])
    m = jnp.maximum(m, s10[pl.ds(1, 56), :, :])
    m = jnp.maximum(m, s11[pl.ds(0, 56), pl.ds(0, 56), :])
    m = jnp.maximum(m, s11[pl.ds(0, 56), pl.ds(1, 56), :])
    m = jnp.maximum(m, s11[pl.ds(1, 56), pl.ds(0, 56), :])
    m = jnp.maximum(m, s11[pl.ds(1, 56), pl.ds(1, 56), :])
    o_ref[...] = m.reshape(56 * 56, 64).astype(jnp.bfloat16)


def _pool(y0q, stats0):
    s0, q0, g0, b0 = stats0
    body = functools.partial(_pool_body, count=float(64 * 112 * 112))

    def q_spec(ph):
        return pl.BlockSpec((1, 1, 56, 56, 64),
                            lambda i: (i, ph, 0, 0, 0))

    return pl.pallas_call(
        body,
        out_shape=jax.ShapeDtypeStruct((64 * 56 * 56, 64), jnp.bfloat16),
        grid_spec=pltpu.PrefetchScalarGridSpec(
            num_scalar_prefetch=0,
            grid=(64,),
            in_specs=[
                q_spec(0), q_spec(1), q_spec(2), q_spec(3),
                pl.BlockSpec((64, 1, 64), lambda i: (0, 0, 0)),
                pl.BlockSpec((64, 1, 64), lambda i: (0, 0, 0)),
                pl.BlockSpec((1, 64), lambda i: (0, 0)),
                pl.BlockSpec((1, 64), lambda i: (0, 0)),
            ],
            out_specs=pl.BlockSpec((56 * 56, 64), lambda i: (i, 0)),
            scratch_shapes=[
                pltpu.VMEM((56, 57, 64), jnp.float32),
                pltpu.VMEM((57, 56, 64), jnp.float32),
                pltpu.VMEM((57, 57, 64), jnp.float32),
            ]),
        compiler_params=pltpu.CompilerParams(
            dimension_semantics=("parallel",)),
    )(y0q, y0q, y0q, y0q, s0, q0, g0.reshape(1, 64).astype(jnp.float32),
      b0.reshape(1, 64).astype(jnp.float32))


def _head_body(x_ref, w_ref, b_ref, o_ref):
    xm = jnp.mean(x_ref[...].astype(jnp.float32), axis=1)
    o_ref[...] = (jnp.dot(xm, w_ref[...],
                          preferred_element_type=jnp.float32) + b_ref[...])


def _head(x4, fcw, fcb):
    x3 = x4.reshape(64, 49, 512)
    wT = jnp.transpose(fcw).astype(jnp.float32)
    b2 = fcb.reshape(1, 1000).astype(jnp.float32)
    return pl.pallas_call(
        _head_body,
        out_shape=jax.ShapeDtypeStruct((64, 1000), jnp.float32),
        grid_spec=pltpu.PrefetchScalarGridSpec(
            num_scalar_prefetch=0,
            grid=(1,),
            in_specs=[
                pl.BlockSpec((64, 49, 512), lambda i: (0, 0, 0)),
                pl.BlockSpec((512, 1000), lambda i: (0, 0)),
                pl.BlockSpec((1, 1000), lambda i: (0, 0)),
            ],
            out_specs=pl.BlockSpec((64, 1000), lambda i: (0, 0))),
        compiler_params=pltpu.CompilerParams(
            dimension_semantics=("arbitrary",)),
    )(x3, wT, b2)


def _w9(w):
    OC, IC, KH, KW = w.shape
    return jnp.transpose(w, (2, 3, 1, 0)).reshape(KH * KW, IC, OC).astype(
        jnp.bfloat16)


def _basic_block(x2, p, *, H, C, OC, stride, NB_in, NB_out):
    OH = H // stride
    count_o = float(64 * OH * OH)
    wd = p.get('down_w')
    if stride == 2:
        ph = _phases(x2, H, C, pad=True)
        y1, s1, q1, *down = _conv_s2(
            ph, _w9(p['conv1_w']), _w9(wd), NB=NB_in, H=H, C=C, OC=OC)
    else:
        y1, s1, q1 = _conv(
            x2, _w9(p['conv1_w']), NB=NB_in, H=H, W=H, C=C, OC=OC)
        down = []
    stats1 = (s1, q1, p['bn1_g'], p['bn1_b'])
    y2, s2, q2 = _conv(
        y1, _w9(p['conv2_w']), NB=NB_out, H=OH, W=OH, C=OC, OC=OC,
        stats_in=stats1)
    stats2 = (s2, q2, p['bn2_g'], p['bn2_b'])
    if wd is not None:
        yd, sd, qd = down
        statsd = (sd, qd, p['down_bn_g'], p['down_bn_b'])
        res = yd
    else:
        statsd = None
        res = x2
    G = 64 // NB_out
    return _residual(y2, stats2, res, statsd,
                     rows=64 * OH * OH, C=OC, G=G,
                     count=count_o, count_d=count_o)


def kernel(x, c1w, b1g, b1b, l1_0_c1w, l1_0_b1g, l1_0_b1b, l1_0_c2w, l1_0_b2g, l1_0_b2b, l1_1_c1w, l1_1_b1g, l1_1_b1b, l1_1_c2w, l1_1_b2g, l1_1_b2b, l2_0_c1w, l2_0_b1g, l2_0_b1b, l2_0_c2w, l2_0_b2g, l2_0_b2b, l2_0_dw, l2_0_dbg, l2_0_dbb, l2_1_c1w, l2_1_b1g, l2_1_b1b, l2_1_c2w, l2_1_b2g, l2_1_b2b, l3_0_c1w, l3_0_b1g, l3_0_b1b, l3_0_c2w, l3_0_b2g, l3_0_b2b, l3_0_dw, l3_0_dbg, l3_0_dbb, l3_1_c1w, l3_1_b1g, l3_1_b1b, l3_1_c2w, l3_1_b2g, l3_1_b2b, l4_0_c1w, l4_0_b1g, l4_0_b1b, l4_0_c2w, l4_0_b2g, l4_0_b2b, l4_0_dw, l4_0_dbg, l4_0_dbb, l4_1_c1w, l4_1_b1g, l4_1_b1b, l4_1_c2w, l4_1_b2g, l4_1_b2b, fcw, fcb):
    xp = jnp.pad(x, ((0, 0), (0, 0), (3, 5), (3, 5)))
    xs = xp.reshape(64, 3, 116, 2, 116, 2).transpose(
        0, 2, 4, 3, 5, 1).reshape(64, 116, 116, 12).astype(jnp.bfloat16)
    xq = [xs[:, r::2, c::2, :] for r in (0, 1) for c in (0, 1)]
    w8 = jnp.pad(c1w, ((0, 0), (0, 0), (0, 1), (0, 1)))
    w16 = w8.reshape(64, 3, 4, 2, 4, 2).transpose(
        2, 4, 3, 5, 1, 0).reshape(16, 12, 64).astype(jnp.bfloat16)
    y0q, s0, q0 = _stem(xq, w16)
    x1 = _pool(y0q, (s0, q0, b1g, b1b))

    p_l1_0 = {'conv1_w': l1_0_c1w, 'bn1_g': l1_0_b1g, 'bn1_b': l1_0_b1b,
              'conv2_w': l1_0_c2w, 'bn2_g': l1_0_b2g, 'bn2_b': l1_0_b2b}
    p_l1_1 = {'conv1_w': l1_1_c1w, 'bn1_g': l1_1_b1g, 'bn1_b': l1_1_b1b,
              'conv2_w': l1_1_c2w, 'bn2_g': l1_1_b2g, 'bn2_b': l1_1_b2b}
    p_l2_0 = {'conv1_w': l2_0_c1w, 'bn1_g': l2_0_b1g, 'bn1_b': l2_0_b1b,
              'conv2_w': l2_0_c2w, 'bn2_g': l2_0_b2g, 'bn2_b': l2_0_b2b,
              'down_w': l2_0_dw, 'down_bn_g': l2_0_dbg, 'down_bn_b': l2_0_dbb}
    p_l2_1 = {'conv1_w': l2_1_c1w, 'bn1_g': l2_1_b1g, 'bn1_b': l2_1_b1b,
              'conv2_w': l2_1_c2w, 'bn2_g': l2_1_b2g, 'bn2_b': l2_1_b2b}
    p_l3_0 = {'conv1_w': l3_0_c1w, 'bn1_g': l3_0_b1g, 'bn1_b': l3_0_b1b,
              'conv2_w': l3_0_c2w, 'bn2_g': l3_0_b2g, 'bn2_b': l3_0_b2b,
              'down_w': l3_0_dw, 'down_bn_g': l3_0_dbg, 'down_bn_b': l3_0_dbb}
    p_l3_1 = {'conv1_w': l3_1_c1w, 'bn1_g': l3_1_b1g, 'bn1_b': l3_1_b1b,
              'conv2_w': l3_1_c2w, 'bn2_g': l3_1_b2g, 'bn2_b': l3_1_b2b}
    p_l4_0 = {'conv1_w': l4_0_c1w, 'bn1_g': l4_0_b1g, 'bn1_b': l4_0_b1b,
              'conv2_w': l4_0_c2w, 'bn2_g': l4_0_b2g, 'bn2_b': l4_0_b2b,
              'down_w': l4_0_dw, 'down_bn_g': l4_0_dbg, 'down_bn_b': l4_0_dbb}
    p_l4_1 = {'conv1_w': l4_1_c1w, 'bn1_g': l4_1_b1g, 'bn1_b': l4_1_b1b,
              'conv2_w': l4_1_c2w, 'bn2_g': l4_1_b2g, 'bn2_b': l4_1_b2b}

    x = _basic_block(x1, p_l1_0, H=56, C=64, OC=64, stride=1,
                     NB_in=1, NB_out=1)
    x = _basic_block(x, p_l1_1, H=56, C=64, OC=64, stride=1,
                     NB_in=1, NB_out=1)
    x = _basic_block(x, p_l2_0, H=56, C=64, OC=128, stride=2,
                     NB_in=4, NB_out=4)
    x = _basic_block(x, p_l2_1, H=28, C=128, OC=128, stride=1,
                     NB_in=4, NB_out=4)
    x = _basic_block(x, p_l3_0, H=28, C=128, OC=256, stride=2,
                     NB_in=8, NB_out=8)
    x = _basic_block(x, p_l3_1, H=14, C=256, OC=256, stride=1,
                     NB_in=8, NB_out=8)
    x = _basic_block(x, p_l4_0, H=14, C=256, OC=512, stride=2,
                     NB_in=16, NB_out=16)
    x = _basic_block(x, p_l4_1, H=7, C=512, OC=512, stride=1,
                     NB_in=16, NB_out=16)

    return _head(x, fcw, fcb)


# speedup vs baseline: 5.8314x; 1.0462x over previous
import functools

import jax
import jax.numpy as jnp
from jax.experimental import pallas as pl
from jax.experimental.pallas import tpu as pltpu

_EPS = 1e-5


def _bn_coeffs(sum_ref, ssq_ref, g_ref, b_ref, count):
    s = jnp.sum(sum_ref[...], axis=0)
    q = jnp.sum(ssq_ref[...], axis=0)
    inv = 1.0 / count
    mean = s * inv
    var = jnp.maximum(q * inv - mean * mean, 0.0)
    scale = g_ref[...] * jax.lax.rsqrt(var + _EPS)
    shift = b_ref[...] - mean * scale
    return scale, shift


def _conv_body(*refs, NB, H, W, C, OC, has_bn, count_in):
    it = iter(refs)
    x_ref = next(it)
    if has_bn:
        psum, pssq, g_ref, b_ref = next(it), next(it), next(it), next(it)
    w_ref = next(it)
    o_ref, os_ref, oq_ref = next(it), next(it), next(it)
    pr = next(it)

    M = NB * H * W
    xb = x_ref[...]
    if has_bn:
        scale, shift = _bn_coeffs(psum, pssq, g_ref, b_ref, count_in)
        a = jnp.maximum(xb.astype(jnp.float32) * scale + shift, 0.0)
        a = a.astype(jnp.bfloat16)
    else:
        a = xb

    pr[...] = jnp.zeros_like(pr)
    pr[:, pl.ds(1, H), pl.ds(1, W), :] = a.reshape(NB, H, W, C)

    acc = None
    for dy in range(3):
        for dx in range(3):
            sl = pr[:, pl.ds(dy, H), pl.ds(dx, W), :]
            at = sl.reshape(M, C)
            d = jnp.dot(at, w_ref[dy * 3 + dx],
                        preferred_element_type=jnp.float32)
            acc = d if acc is None else acc + d

    o_ref[...] = acc.astype(jnp.bfloat16)
    os_ref[...] = jnp.sum(acc, axis=0, keepdims=True)[None]
    oq_ref[...] = jnp.sum(acc * acc, axis=0, keepdims=True)[None]


def _conv_s2_body(p00, p01, p10, p11, w_ref, wd_ref, o_ref, os_ref, oq_ref,
                  od_ref, ods_ref, odq_ref, *, NB, OH, C):
    ph = (p00, p01, p10, p11)
    M = NB * OH * OH
    acc = None
    for dy in range(3):
        for dx in range(3):
            ref = ph[(dy % 2) * 2 + (dx % 2)]
            sl = ref[:, pl.ds(dy // 2, OH), pl.ds(dx // 2, OH), :]
            at = sl.reshape(M, C)
            d = jnp.dot(at, w_ref[dy * 3 + dx],
                        preferred_element_type=jnp.float32)
            acc = d if acc is None else acc + d
    o_ref[...] = acc.astype(jnp.bfloat16)
    os_ref[...] = jnp.sum(acc, axis=0, keepdims=True)[None]
    oq_ref[...] = jnp.sum(acc * acc, axis=0, keepdims=True)[None]

    ad = p11[:, pl.ds(0, OH), pl.ds(0, OH), :]
    accd = jnp.dot(ad.reshape(M, C), wd_ref[0],
                   preferred_element_type=jnp.float32)
    od_ref[...] = accd.astype(jnp.bfloat16)
    ods_ref[...] = jnp.sum(accd, axis=0, keepdims=True)[None]
    odq_ref[...] = jnp.sum(accd * accd, axis=0, keepdims=True)[None]


def _stat_specs(G, OC):
    return [
        pl.BlockSpec((1, 1, OC), lambda i: (i, 0, 0)),
        pl.BlockSpec((1, 1, OC), lambda i: (i, 0, 0)),
    ]


def _stat_shapes(G, OC):
    return [
        jax.ShapeDtypeStruct((G, 1, OC), jnp.float32),
        jax.ShapeDtypeStruct((G, 1, OC), jnp.float32),
    ]


def _conv(x2, w9, *, NB, H, W, C, OC, stats_in=None):
    N = 64
    G = N // NB
    M = NB * H * W
    has_bn = stats_in is not None

    in_specs = [pl.BlockSpec((M, C), lambda i: (i, 0))]
    args = [x2]
    if has_bn:
        s_in, q_in, g_in, b_in = stats_in
        gp = s_in.shape[0]
        in_specs += [
            pl.BlockSpec((gp, 1, C), lambda i: (0, 0, 0)),
            pl.BlockSpec((gp, 1, C), lambda i: (0, 0, 0)),
            pl.BlockSpec((1, C), lambda i: (0, 0)),
            pl.BlockSpec((1, C), lambda i: (0, 0)),
        ]
        args += [s_in, q_in, g_in.reshape(1, C).astype(jnp.float32),
                 b_in.reshape(1, C).astype(jnp.float32)]
    in_specs.append(pl.BlockSpec((9, C, OC), lambda i: (0, 0, 0)))
    args.append(w9)

    body = functools.partial(_conv_body, NB=NB, H=H, W=W, C=C, OC=OC,
                             has_bn=has_bn, count_in=float(N * H * W))

    return pl.pallas_call(
        body,
        out_shape=[jax.ShapeDtypeStruct((N * H * W, OC), jnp.bfloat16)]
        + _stat_shapes(G, OC),
        grid_spec=pltpu.PrefetchScalarGridSpec(
            num_scalar_prefetch=0,
            grid=(G,),
            in_specs=in_specs,
            out_specs=[pl.BlockSpec((M, OC), lambda i: (i, 0))]
            + _stat_specs(G, OC),
            scratch_shapes=[
                pltpu.VMEM((NB, H + 2, W + 2, C), jnp.bfloat16)]),
        compiler_params=pltpu.CompilerParams(
            dimension_semantics=("parallel",)),
    )(*args)


def _phases(x2, H, C, pad):
    x4 = x2.reshape(64, H, H, C)
    if pad:
        x4 = jnp.pad(x4, ((0, 0), (1, 1), (1, 1), (0, 0)))
    return [x4[:, r::2, s::2, :] for r in (0, 1) for s in (0, 1)]


def _conv_s2(ph, w9, wd, *, NB, H, C, OC):
    N = 64
    G = N // NB
    OH = H // 2
    PH = (H + 2) // 2
    M = NB * OH * OH
    OCD = wd.shape[2]

    ph_spec = pl.BlockSpec((NB, PH, PH, C), lambda i: (i, 0, 0, 0))
    body = functools.partial(_conv_s2_body, NB=NB, OH=OH, C=C)
    return pl.pallas_call(
        body,
        out_shape=[jax.ShapeDtypeStruct((N * OH * OH, OC), jnp.bfloat16)]
        + _stat_shapes(G, OC)
        + [jax.ShapeDtypeStruct((N * OH * OH, OCD), jnp.bfloat16)]
        + _stat_shapes(G, OCD),
        grid_spec=pltpu.PrefetchScalarGridSpec(
            num_scalar_prefetch=0,
            grid=(G,),
            in_specs=[ph_spec, ph_spec, ph_spec, ph_spec,
                      pl.BlockSpec((9, C, OC), lambda i: (0, 0, 0)),
                      pl.BlockSpec((1, C, OCD), lambda i: (0, 0, 0))],
            out_specs=[pl.BlockSpec((M, OC), lambda i: (i, 0))]
            + _stat_specs(G, OC)
            + [pl.BlockSpec((M, OCD), lambda i: (i, 0))]
            + _stat_specs(G, OCD)),
        compiler_params=pltpu.CompilerParams(
            dimension_semantics=("parallel",)),
    )(*ph, w9, wd)


def _residual_body(y_ref, ys_ref, yq_ref, yg_ref, yb_ref, r_ref, *rest,
                   count, count_d, has_dstats):
    if has_dstats:
        rs_ref, rq_ref, rg_ref, rb_ref, o_ref = rest
    else:
        (o_ref,) = rest
    scale, shift = _bn_coeffs(ys_ref, yq_ref, yg_ref, yb_ref, count)
    y = y_ref[...].astype(jnp.float32) * scale + shift
    if has_dstats:
        ds, dh = _bn_coeffs(rs_ref, rq_ref, rg_ref, rb_ref, count_d)
        r = r_ref[...].astype(jnp.float32) * ds + dh
    else:
        r = r_ref[...].astype(jnp.float32)
    o_ref[...] = jnp.maximum(y + r, 0.0).astype(jnp.bfloat16)


def _residual(y2, stats2, res2, statsd, *, rows, C, G, count, count_d):
    TR = rows // G
    s2, q2, g2, b2 = stats2
    gp = s2.shape[0]
    row_spec = pl.BlockSpec((TR, C), lambda i: (i, 0))
    st_spec = pl.BlockSpec((gp, 1, C), lambda i: (0, 0, 0))
    vec_spec = pl.BlockSpec((1, C), lambda i: (0, 0))
    in_specs = [row_spec, st_spec, st_spec, vec_spec, vec_spec, row_spec]
    args = [y2, s2, q2, g2.reshape(1, C).astype(jnp.float32),
            b2.reshape(1, C).astype(jnp.float32), res2]
    if statsd is not None:
        sd, qd, gd, bd = statsd
        gpd = sd.shape[0]
        std_spec = pl.BlockSpec((gpd, 1, C), lambda i: (0, 0, 0))
        in_specs += [std_spec, std_spec, vec_spec, vec_spec]
        args += [sd, qd, gd.reshape(1, C).astype(jnp.float32),
                 bd.reshape(1, C).astype(jnp.float32)]
    body = functools.partial(_residual_body, count=count, count_d=count_d,
                             has_dstats=statsd is not None)
    return pl.pallas_call(
        body,
        out_shape=jax.ShapeDtypeStruct((rows, C), jnp.bfloat16),
        grid_spec=pltpu.PrefetchScalarGridSpec(
            num_scalar_prefetch=0,
            grid=(G,),
            in_specs=in_specs,
            out_specs=row_spec),
        compiler_params=pltpu.CompilerParams(
            dimension_semantics=("parallel",)),
    )(*args)


def _stem_body(q00, q01, q10, q11, w_ref, o_ref, os_ref, oq_ref, acc_ref):
    qs = (q00, q01, q10, q11)
    ssum = None
    for r in (0, 1):
        for c in (0, 1):
            for a in range(4):
                for b in range(4):
                    u, v = r + a, c + b
                    ref = qs[(u % 2) * 2 + (v % 2)]
                    sl = ref[0, 0, pl.ds(u // 2, 56), pl.ds(v // 2, 56), :]
                    at = sl.reshape(56 * 56, 12)
                    d = jnp.dot(at, w_ref[a * 4 + b],
                                preferred_element_type=jnp.float32)
                    if a == 0 and b == 0:
                        acc_ref[...] = d
                    else:
                        acc_ref[...] += d
            acc = acc_ref[...]
            o_ref[0, r * 2 + c] = acc.reshape(56, 56, 64).astype(jnp.bfloat16)
            s1 = jnp.sum(acc, axis=0, keepdims=True)
            s2 = jnp.sum(acc * acc, axis=0, keepdims=True)
            ssum = (s1, s2) if ssum is None else (ssum[0] + s1, ssum[1] + s2)
    os_ref[...] = ssum[0][None]
    oq_ref[...] = ssum[1][None]


def _stem(xqs, w16):
    def q_spec(ph):
        return pl.BlockSpec((1, 1, 58, 58, 12),
                            lambda i: (i, ph, 0, 0, 0))
    return pl.pallas_call(
        _stem_body,
        out_shape=[
            jax.ShapeDtypeStruct((64, 4, 56, 56, 64), jnp.bfloat16),
            jax.ShapeDtypeStruct((64, 1, 64), jnp.float32),
            jax.ShapeDtypeStruct((64, 1, 64), jnp.float32),
        ],
        grid_spec=pltpu.PrefetchScalarGridSpec(
            num_scalar_prefetch=0,
            grid=(64,),
            in_specs=[q_spec(0), q_spec(1), q_spec(2), q_spec(3),
                      pl.BlockSpec((16, 12, 64), lambda i: (0, 0, 0))],
            out_specs=[
                pl.BlockSpec((1, 4, 56, 56, 64), lambda i: (i, 0, 0, 0, 0)),
                pl.BlockSpec((1, 1, 64), lambda i: (i, 0, 0)),
                pl.BlockSpec((1, 1, 64), lambda i: (i, 0, 0)),
            ],
            scratch_shapes=[pltpu.VMEM((56 * 56, 64), jnp.float32)]),
        compiler_params=pltpu.CompilerParams(
            dimension_semantics=("parallel",)),
    )(xqs, xqs, xqs, xqs, w16)


def _pool_body(q00, q01, q10, q11, psum, pssq, g_ref, b_ref, o_ref,
               s01, s10, s11, *, count):
    scale, shift = _bn_coeffs(psum, pssq, g_ref, b_ref, count)

    def bn(qref):
        v = qref[0, 0].astype(jnp.float32)
        return jnp.maximum(v * scale + shift, 0.0)

    s01[...] = jnp.full_like(s01, -jnp.inf)
    s01[:, pl.ds(1, 56), :] = bn(q01)
    s10[...] = jnp.full_like(s10, -jnp.inf)
    s10[pl.ds(1, 56), :, :] = bn(q10)
    s11[...] = jnp.full_like(s11, -jnp.inf)
    s11[pl.ds(1, 56), pl.ds(1, 56), :] = bn(q11)

    m = bn(q00)
    m = jnp.maximum(m, s01[:, pl.ds(0, 56), :])
    m = jnp.maximum(m, s01[:, pl.ds(1, 56), :])
    m = jnp.maximum(m, s10[pl.ds(0, 56), :, :])
    m = jnp.maximum(m, s10[pl.ds(1, 56), :, :])
    m = jnp.maximum(m, s11[pl.ds(0, 56), pl.ds(0, 56), :])
    m = jnp.maximum(m, s11[pl.ds(0, 56), pl.ds(1, 56), :])
    m = jnp.maximum(m, s11[pl.ds(1, 56), pl.ds(0, 56), :])
    m = jnp.maximum(m, s11[pl.ds(1, 56), pl.ds(1, 56), :])
    o_ref[...] = m.reshape(56 * 56, 64).astype(jnp.bfloat16)


def _pool(y0q, stats0):
    s0, q0, g0, b0 = stats0
    body = functools.partial(_pool_body, count=float(64 * 112 * 112))

    def q_spec(ph):
        return pl.BlockSpec((1, 1, 56, 56, 64),
                            lambda i: (i, ph, 0, 0, 0))

    return pl.pallas_call(
        body,
        out_shape=jax.ShapeDtypeStruct((64 * 56 * 56, 64), jnp.bfloat16),
        grid_spec=pltpu.PrefetchScalarGridSpec(
            num_scalar_prefetch=0,
            grid=(64,),
            in_specs=[
                q_spec(0), q_spec(1), q_spec(2), q_spec(3),
                pl.BlockSpec((64, 1, 64), lambda i: (0, 0, 0)),
                pl.BlockSpec((64, 1, 64), lambda i: (0, 0, 0)),
                pl.BlockSpec((1, 64), lambda i: (0, 0)),
                pl.BlockSpec((1, 64), lambda i: (0, 0)),
            ],
            out_specs=pl.BlockSpec((56 * 56, 64), lambda i: (i, 0)),
            scratch_shapes=[
                pltpu.VMEM((56, 57, 64), jnp.float32),
                pltpu.VMEM((57, 56, 64), jnp.float32),
                pltpu.VMEM((57, 57, 64), jnp.float32),
            ]),
        compiler_params=pltpu.CompilerParams(
            dimension_semantics=("parallel",)),
    )(y0q, y0q, y0q, y0q, s0, q0, g0.reshape(1, 64).astype(jnp.float32),
      b0.reshape(1, 64).astype(jnp.float32))


def _head_body(x_ref, w_ref, b_ref, o_ref):
    xm = jnp.mean(x_ref[...].astype(jnp.float32), axis=1)
    o_ref[...] = (jnp.dot(xm, w_ref[...],
                          preferred_element_type=jnp.float32) + b_ref[...])


def _head(x4, fcw, fcb):
    x3 = x4.reshape(64, 49, 512)
    wT = jnp.transpose(fcw).astype(jnp.float32)
    b2 = fcb.reshape(1, 1000).astype(jnp.float32)
    return pl.pallas_call(
        _head_body,
        out_shape=jax.ShapeDtypeStruct((64, 1000), jnp.float32),
        grid_spec=pltpu.PrefetchScalarGridSpec(
            num_scalar_prefetch=0,
            grid=(1,),
            in_specs=[
                pl.BlockSpec((64, 49, 512), lambda i: (0, 0, 0)),
                pl.BlockSpec((512, 1000), lambda i: (0, 0)),
                pl.BlockSpec((1, 1000), lambda i: (0, 0)),
            ],
            out_specs=pl.BlockSpec((64, 1000), lambda i: (0, 0))),
        compiler_params=pltpu.CompilerParams(
            dimension_semantics=("arbitrary",)),
    )(x3, wT, b2)


def _w9(w):
    OC, IC, KH, KW = w.shape
    return jnp.transpose(w, (2, 3, 1, 0)).reshape(KH * KW, IC, OC).astype(
        jnp.bfloat16)


def _basic_block(x2, p, *, H, C, OC, stride, NB_in, NB_out):
    OH = H // stride
    count_o = float(64 * OH * OH)
    wd = p.get('down_w')
    if stride == 2:
        ph = _phases(x2, H, C, pad=True)
        y1, s1, q1, *down = _conv_s2(
            ph, _w9(p['conv1_w']), _w9(wd), NB=NB_in, H=H, C=C, OC=OC)
    else:
        y1, s1, q1 = _conv(
            x2, _w9(p['conv1_w']), NB=NB_in, H=H, W=H, C=C, OC=OC)
        down = []
    stats1 = (s1, q1, p['bn1_g'], p['bn1_b'])
    y2, s2, q2 = _conv(
        y1, _w9(p['conv2_w']), NB=NB_out, H=OH, W=OH, C=OC, OC=OC,
        stats_in=stats1)
    stats2 = (s2, q2, p['bn2_g'], p['bn2_b'])
    if wd is not None:
        yd, sd, qd = down
        statsd = (sd, qd, p['down_bn_g'], p['down_bn_b'])
        res = yd
    else:
        statsd = None
        res = x2
    G = 64 // NB_out
    return _residual(y2, stats2, res, statsd,
                     rows=64 * OH * OH, C=OC, G=G,
                     count=count_o, count_d=count_o)


def kernel(x, c1w, b1g, b1b, l1_0_c1w, l1_0_b1g, l1_0_b1b, l1_0_c2w, l1_0_b2g, l1_0_b2b, l1_1_c1w, l1_1_b1g, l1_1_b1b, l1_1_c2w, l1_1_b2g, l1_1_b2b, l2_0_c1w, l2_0_b1g, l2_0_b1b, l2_0_c2w, l2_0_b2g, l2_0_b2b, l2_0_dw, l2_0_dbg, l2_0_dbb, l2_1_c1w, l2_1_b1g, l2_1_b1b, l2_1_c2w, l2_1_b2g, l2_1_b2b, l3_0_c1w, l3_0_b1g, l3_0_b1b, l3_0_c2w, l3_0_b2g, l3_0_b2b, l3_0_dw, l3_0_dbg, l3_0_dbb, l3_1_c1w, l3_1_b1g, l3_1_b1b, l3_1_c2w, l3_1_b2g, l3_1_b2b, l4_0_c1w, l4_0_b1g, l4_0_b1b, l4_0_c2w, l4_0_b2g, l4_0_b2b, l4_0_dw, l4_0_dbg, l4_0_dbb, l4_1_c1w, l4_1_b1g, l4_1_b1b, l4_1_c2w, l4_1_b2g, l4_1_b2b, fcw, fcb):
    xp = jnp.pad(x, ((0, 0), (0, 0), (3, 5), (3, 5)))
    xqs = xp.reshape(64, 3, 58, 2, 2, 58, 2, 2).transpose(
        0, 3, 6, 2, 5, 4, 7, 1).reshape(64, 4, 58, 58, 12).astype(
        jnp.bfloat16)
    w8 = jnp.pad(c1w, ((0, 0), (0, 0), (0, 1), (0, 1)))
    w16 = w8.reshape(64, 3, 4, 2, 4, 2).transpose(
        2, 4, 3, 5, 1, 0).reshape(16, 12, 64).astype(jnp.bfloat16)
    y0q, s0, q0 = _stem(xqs, w16)
    x1 = _pool(y0q, (s0, q0, b1g, b1b))

    p_l1_0 = {'conv1_w': l1_0_c1w, 'bn1_g': l1_0_b1g, 'bn1_b': l1_0_b1b,
              'conv2_w': l1_0_c2w, 'bn2_g': l1_0_b2g, 'bn2_b': l1_0_b2b}
    p_l1_1 = {'conv1_w': l1_1_c1w, 'bn1_g': l1_1_b1g, 'bn1_b': l1_1_b1b,
              'conv2_w': l1_1_c2w, 'bn2_g': l1_1_b2g, 'bn2_b': l1_1_b2b}
    p_l2_0 = {'conv1_w': l2_0_c1w, 'bn1_g': l2_0_b1g, 'bn1_b': l2_0_b1b,
              'conv2_w': l2_0_c2w, 'bn2_g': l2_0_b2g, 'bn2_b': l2_0_b2b,
              'down_w': l2_0_dw, 'down_bn_g': l2_0_dbg, 'down_bn_b': l2_0_dbb}
    p_l2_1 = {'conv1_w': l2_1_c1w, 'bn1_g': l2_1_b1g, 'bn1_b': l2_1_b1b,
              'conv2_w': l2_1_c2w, 'bn2_g': l2_1_b2g, 'bn2_b': l2_1_b2b}
    p_l3_0 = {'conv1_w': l3_0_c1w, 'bn1_g': l3_0_b1g, 'bn1_b': l3_0_b1b,
              'conv2_w': l3_0_c2w, 'bn2_g': l3_0_b2g, 'bn2_b': l3_0_b2b,
              'down_w': l3_0_dw, 'down_bn_g': l3_0_dbg, 'down_bn_b': l3_0_dbb}
    p_l3_1 = {'conv1_w': l3_1_c1w, 'bn1_g': l3_1_b1g, 'bn1_b': l3_1_b1b,
              'conv2_w': l3_1_c2w, 'bn2_g': l3_1_b2g, 'bn2_b': l3_1_b2b}
    p_l4_0 = {'conv1_w': l4_0_c1w, 'bn1_g': l4_0_b1g, 'bn1_b': l4_0_b1b,
              'conv2_w': l4_0_c2w, 'bn2_g': l4_0_b2g, 'bn2_b': l4_0_b2b,
              'down_w': l4_0_dw, 'down_bn_g': l4_0_dbg, 'down_bn_b': l4_0_dbb}
    p_l4_1 = {'conv1_w': l4_1_c1w, 'bn1_g': l4_1_b1g, 'bn1_b': l4_1_b1b,
              'conv2_w': l4_1_c2w, 'bn2_g': l4_1_b2g, 'bn2_b': l4_1_b2b}

    x = _basic_block(x1, p_l1_0, H=56, C=64, OC=64, stride=1,
                     NB_in=1, NB_out=1)
    x = _basic_block(x, p_l1_1, H=56, C=64, OC=64, stride=1,
                     NB_in=1, NB_out=1)
    x = _basic_block(x, p_l2_0, H=56, C=64, OC=128, stride=2,
                     NB_in=4, NB_out=4)
    x = _basic_block(x, p_l2_1, H=28, C=128, OC=128, stride=1,
                     NB_in=4, NB_out=4)
    x = _basic_block(x, p_l3_0, H=28, C=128, OC=256, stride=2,
                     NB_in=8, NB_out=8)
    x = _basic_block(x, p_l3_1, H=14, C=256, OC=256, stride=1,
                     NB_in=8, NB_out=8)
    x = _basic_block(x, p_l4_0, H=14, C=256, OC=512, stride=2,
                     NB_in=16, NB_out=16)
    x = _basic_block(x, p_l4_1, H=7, C=512, OC=512, stride=1,
                     NB_in=16, NB_out=16)

    return _head(x, fcw, fcb)


# speedup vs baseline: 5.8639x; 1.0056x over previous
import functools

import jax
import jax.numpy as jnp
from jax.experimental import pallas as pl
from jax.experimental.pallas import tpu as pltpu

_EPS = 1e-5


def _bn_coeffs(sum_ref, ssq_ref, g_ref, b_ref, count):
    s = jnp.sum(sum_ref[...], axis=0)
    q = jnp.sum(ssq_ref[...], axis=0)
    inv = 1.0 / count
    mean = s * inv
    var = jnp.maximum(q * inv - mean * mean, 0.0)
    scale = g_ref[...] * jax.lax.rsqrt(var + _EPS)
    shift = b_ref[...] - mean * scale
    return scale, shift


def _conv_body(*refs, NB, H, W, C, OC, mode, count_in):
    it = iter(refs)
    x_ref = next(it)
    if mode != 'plain':
        psum, pssq, g_ref, b_ref = next(it), next(it), next(it), next(it)
    if mode in ('res', 'res_bn'):
        id_ref = next(it)
    if mode == 'res_bn':
        dsum, dssq, dg_ref, db_ref = next(it), next(it), next(it), next(it)
    w_ref = next(it)
    ores_ref = next(it) if mode in ('res', 'res_bn') else None
    o_ref, os_ref, oq_ref = next(it), next(it), next(it)
    pr = next(it)

    M = NB * H * W
    xb = x_ref[...]
    if mode == 'plain':
        a = xb
    else:
        scale, shift = _bn_coeffs(psum, pssq, g_ref, b_ref, count_in)
        a = xb.astype(jnp.float32) * scale + shift
        if mode == 'res':
            a = a + id_ref[...].astype(jnp.float32)
        elif mode == 'res_bn':
            ds, dh = _bn_coeffs(dsum, dssq, dg_ref, db_ref, count_in)
            a = a + id_ref[...].astype(jnp.float32) * ds + dh
        a = jnp.maximum(a, 0.0).astype(jnp.bfloat16)
    if ores_ref is not None:
        ores_ref[...] = a

    pr[...] = jnp.zeros_like(pr)
    pr[:, pl.ds(1, H), pl.ds(1, W), :] = a.reshape(NB, H, W, C)

    acc = None
    for dy in range(3):
        for dx in range(3):
            sl = pr[:, pl.ds(dy, H), pl.ds(dx, W), :]
            at = sl.reshape(M, C)
            d = jnp.dot(at, w_ref[dy * 3 + dx],
                        preferred_element_type=jnp.float32)
            acc = d if acc is None else acc + d

    o_ref[...] = acc.astype(jnp.bfloat16)
    os_ref[...] = jnp.sum(acc, axis=0, keepdims=True)[None]
    oq_ref[...] = jnp.sum(acc * acc, axis=0, keepdims=True)[None]


def _conv_s2_body(p00, p01, p10, p11, w_ref, wd_ref, o_ref, os_ref, oq_ref,
                  od_ref, ods_ref, odq_ref, *, NB, OH, C):
    ph = (p00, p01, p10, p11)
    M = NB * OH * OH
    acc = None
    for dy in range(3):
        for dx in range(3):
            ref = ph[(dy % 2) * 2 + (dx % 2)]
            sl = ref[:, pl.ds(dy // 2, OH), pl.ds(dx // 2, OH), :]
            at = sl.reshape(M, C)
            d = jnp.dot(at, w_ref[dy * 3 + dx],
                        preferred_element_type=jnp.float32)
            acc = d if acc is None else acc + d
    o_ref[...] = acc.astype(jnp.bfloat16)
    os_ref[...] = jnp.sum(acc, axis=0, keepdims=True)[None]
    oq_ref[...] = jnp.sum(acc * acc, axis=0, keepdims=True)[None]

    ad = p11[:, pl.ds(0, OH), pl.ds(0, OH), :]
    accd = jnp.dot(ad.reshape(M, C), wd_ref[0],
                   preferred_element_type=jnp.float32)
    od_ref[...] = accd.astype(jnp.bfloat16)
    ods_ref[...] = jnp.sum(accd, axis=0, keepdims=True)[None]
    odq_ref[...] = jnp.sum(accd * accd, axis=0, keepdims=True)[None]


def _stat_specs(G, OC):
    return [
        pl.BlockSpec((1, 1, OC), lambda i: (i, 0, 0)),
        pl.BlockSpec((1, 1, OC), lambda i: (i, 0, 0)),
    ]


def _stat_shapes(G, OC):
    return [
        jax.ShapeDtypeStruct((G, 1, OC), jnp.float32),
        jax.ShapeDtypeStruct((G, 1, OC), jnp.float32),
    ]


def _conv(x2, w9, *, NB, H, W, C, OC, stats_in=None, residual=None):
    N = 64
    G = N // NB
    M = NB * H * W
    if stats_in is None:
        mode = 'plain'
    elif residual is None:
        mode = 'bn'
    else:
        mode = 'res' if residual[1] is None else 'res_bn'

    def vecf(v):
        return v.reshape(1, C).astype(jnp.float32)

    def stat_in_specs(gp):
        return [
            pl.BlockSpec((gp, 1, C), lambda i: (0, 0, 0)),
            pl.BlockSpec((gp, 1, C), lambda i: (0, 0, 0)),
            pl.BlockSpec((1, C), lambda i: (0, 0)),
            pl.BlockSpec((1, C), lambda i: (0, 0)),
        ]

    in_specs = [pl.BlockSpec((M, C), lambda i: (i, 0))]
    args = [x2]
    if mode != 'plain':
        s_in, q_in, g_in, b_in = stats_in
        in_specs += stat_in_specs(s_in.shape[0])
        args += [s_in, q_in, vecf(g_in), vecf(b_in)]
    if mode in ('res', 'res_bn'):
        id2, statsd = residual
        in_specs.append(pl.BlockSpec((M, C), lambda i: (i, 0)))
        args.append(id2)
        if statsd is not None:
            sd, qd, gd, bd = statsd
            in_specs += stat_in_specs(sd.shape[0])
            args += [sd, qd, vecf(gd), vecf(bd)]
    in_specs.append(pl.BlockSpec((9, C, OC), lambda i: (0, 0, 0)))
    args.append(w9)

    out_shape = [jax.ShapeDtypeStruct((N * H * W, OC), jnp.bfloat16)]
    out_specs = [pl.BlockSpec((M, OC), lambda i: (i, 0))]
    if mode in ('res', 'res_bn'):
        out_shape = [jax.ShapeDtypeStruct((N * H * W, C), jnp.bfloat16)
                     ] + out_shape
        out_specs = [pl.BlockSpec((M, C), lambda i: (i, 0))] + out_specs

    body = functools.partial(_conv_body, NB=NB, H=H, W=W, C=C, OC=OC,
                             mode=mode, count_in=float(N * H * W))

    return pl.pallas_call(
        body,
        out_shape=out_shape + _stat_shapes(G, OC),
        grid_spec=pltpu.PrefetchScalarGridSpec(
            num_scalar_prefetch=0,
            grid=(G,),
            in_specs=in_specs,
            out_specs=out_specs + _stat_specs(G, OC),
            scratch_shapes=[
                pltpu.VMEM((NB, H + 2, W + 2, C), jnp.bfloat16)]),
        compiler_params=pltpu.CompilerParams(
            dimension_semantics=("parallel",)),
    )(*args)


def _phases(x2, H, C, pad):
    x4 = x2.reshape(64, H, H, C)
    if pad:
        x4 = jnp.pad(x4, ((0, 0), (1, 1), (1, 1), (0, 0)))
    return [x4[:, r::2, s::2, :] for r in (0, 1) for s in (0, 1)]


def _conv_s2(ph, w9, wd, *, NB, H, C, OC):
    N = 64
    G = N // NB
    OH = H // 2
    PH = (H + 2) // 2
    M = NB * OH * OH
    OCD = wd.shape[2]

    ph_spec = pl.BlockSpec((NB, PH, PH, C), lambda i: (i, 0, 0, 0))
    body = functools.partial(_conv_s2_body, NB=NB, OH=OH, C=C)
    return pl.pallas_call(
        body,
        out_shape=[jax.ShapeDtypeStruct((N * OH * OH, OC), jnp.bfloat16)]
        + _stat_shapes(G, OC)
        + [jax.ShapeDtypeStruct((N * OH * OH, OCD), jnp.bfloat16)]
        + _stat_shapes(G, OCD),
        grid_spec=pltpu.PrefetchScalarGridSpec(
            num_scalar_prefetch=0,
            grid=(G,),
            in_specs=[ph_spec, ph_spec, ph_spec, ph_spec,
                      pl.BlockSpec((9, C, OC), lambda i: (0, 0, 0)),
                      pl.BlockSpec((1, C, OCD), lambda i: (0, 0, 0))],
            out_specs=[pl.BlockSpec((M, OC), lambda i: (i, 0))]
            + _stat_specs(G, OC)
            + [pl.BlockSpec((M, OCD), lambda i: (i, 0))]
            + _stat_specs(G, OCD)),
        compiler_params=pltpu.CompilerParams(
            dimension_semantics=("parallel",)),
    )(*ph, w9, wd)


def _residual_body(y_ref, ys_ref, yq_ref, yg_ref, yb_ref, r_ref, *rest,
                   count, count_d, has_dstats):
    if has_dstats:
        rs_ref, rq_ref, rg_ref, rb_ref, o_ref = rest
    else:
        (o_ref,) = rest
    scale, shift = _bn_coeffs(ys_ref, yq_ref, yg_ref, yb_ref, count)
    y = y_ref[...].astype(jnp.float32) * scale + shift
    if has_dstats:
        ds, dh = _bn_coeffs(rs_ref, rq_ref, rg_ref, rb_ref, count_d)
        r = r_ref[...].astype(jnp.float32) * ds + dh
    else:
        r = r_ref[...].astype(jnp.float32)
    o_ref[...] = jnp.maximum(y + r, 0.0).astype(jnp.bfloat16)


def _residual(y2, stats2, res2, statsd, *, rows, C, G, count, count_d):
    TR = rows // G
    s2, q2, g2, b2 = stats2
    gp = s2.shape[0]
    row_spec = pl.BlockSpec((TR, C), lambda i: (i, 0))
    st_spec = pl.BlockSpec((gp, 1, C), lambda i: (0, 0, 0))
    vec_spec = pl.BlockSpec((1, C), lambda i: (0, 0))
    in_specs = [row_spec, st_spec, st_spec, vec_spec, vec_spec, row_spec]
    args = [y2, s2, q2, g2.reshape(1, C).astype(jnp.float32),
            b2.reshape(1, C).astype(jnp.float32), res2]
    if statsd is not None:
        sd, qd, gd, bd = statsd
        gpd = sd.shape[0]
        std_spec = pl.BlockSpec((gpd, 1, C), lambda i: (0, 0, 0))
        in_specs += [std_spec, std_spec, vec_spec, vec_spec]
        args += [sd, qd, gd.reshape(1, C).astype(jnp.float32),
                 bd.reshape(1, C).astype(jnp.float32)]
    body = functools.partial(_residual_body, count=count, count_d=count_d,
                             has_dstats=statsd is not None)
    return pl.pallas_call(
        body,
        out_shape=jax.ShapeDtypeStruct((rows, C), jnp.bfloat16),
        grid_spec=pltpu.PrefetchScalarGridSpec(
            num_scalar_prefetch=0,
            grid=(G,),
            in_specs=in_specs,
            out_specs=row_spec),
        compiler_params=pltpu.CompilerParams(
            dimension_semantics=("parallel",)),
    )(*args)


def _stem_body(q00, q01, q10, q11, w_ref, o_ref, os_ref, oq_ref, acc_ref):
    qs = (q00, q01, q10, q11)
    ssum = None
    for r in (0, 1):
        for c in (0, 1):
            for a in range(4):
                for b in range(4):
                    u, v = r + a, c + b
                    ref = qs[(u % 2) * 2 + (v % 2)]
                    sl = ref[0, 0, pl.ds(u // 2, 56), pl.ds(v // 2, 56), :]
                    at = sl.reshape(56 * 56, 12)
                    d = jnp.dot(at, w_ref[a * 4 + b],
                                preferred_element_type=jnp.float32)
                    if a == 0 and b == 0:
                        acc_ref[...] = d
                    else:
                        acc_ref[...] += d
            acc = acc_ref[...]
            o_ref[0, r * 2 + c] = acc.reshape(56, 56, 64).astype(jnp.bfloat16)
            s1 = jnp.sum(acc, axis=0, keepdims=True)
            s2 = jnp.sum(acc * acc, axis=0, keepdims=True)
            ssum = (s1, s2) if ssum is None else (ssum[0] + s1, ssum[1] + s2)
    os_ref[...] = ssum[0][None]
    oq_ref[...] = ssum[1][None]


def _stem(xqs, w16):
    def q_spec(ph):
        return pl.BlockSpec((1, 1, 58, 58, 12),
                            lambda i: (i, ph, 0, 0, 0))
    return pl.pallas_call(
        _stem_body,
        out_shape=[
            jax.ShapeDtypeStruct((64, 4, 56, 56, 64), jnp.bfloat16),
            jax.ShapeDtypeStruct((64, 1, 64), jnp.float32),
            jax.ShapeDtypeStruct((64, 1, 64), jnp.float32),
        ],
        grid_spec=pltpu.PrefetchScalarGridSpec(
            num_scalar_prefetch=0,
            grid=(64,),
            in_specs=[q_spec(0), q_spec(1), q_spec(2), q_spec(3),
                      pl.BlockSpec((16, 12, 64), lambda i: (0, 0, 0))],
            out_specs=[
                pl.BlockSpec((1, 4, 56, 56, 64), lambda i: (i, 0, 0, 0, 0)),
                pl.BlockSpec((1, 1, 64), lambda i: (i, 0, 0)),
                pl.BlockSpec((1, 1, 64), lambda i: (i, 0, 0)),
            ],
            scratch_shapes=[pltpu.VMEM((56 * 56, 64), jnp.float32)]),
        compiler_params=pltpu.CompilerParams(
            dimension_semantics=("parallel",)),
    )(xqs, xqs, xqs, xqs, w16)


def _pool_body(q00, q01, q10, q11, psum, pssq, g_ref, b_ref, o_ref,
               s01, s10, s11, *, count):
    scale, shift = _bn_coeffs(psum, pssq, g_ref, b_ref, count)

    def bn(qref):
        v = qref[0, 0].astype(jnp.float32)
        return jnp.maximum(v * scale + shift, 0.0)

    s01[...] = jnp.full_like(s01, -jnp.inf)
    s01[:, pl.ds(1, 56), :] = bn(q01)
    s10[...] = jnp.full_like(s10, -jnp.inf)
    s10[pl.ds(1, 56), :, :] = bn(q10)
    s11[...] = jnp.full_like(s11, -jnp.inf)
    s11[pl.ds(1, 56), pl.ds(1, 56), :] = bn(q11)

    m = bn(q00)
    m = jnp.maximum(m, s01[:, pl.ds(0, 56), :])
    m = jnp.maximum(m, s01[:, pl.ds(1, 56), :])
    m = jnp.maximum(m, s10[pl.ds(0, 56), :, :])
    m = jnp.maximum(m, s10[pl.ds(1, 56), :, :])
    m = jnp.maximum(m, s11[pl.ds(0, 56), pl.ds(0, 56), :])
    m = jnp.maximum(m, s11[pl.ds(0, 56), pl.ds(1, 56), :])
    m = jnp.maximum(m, s11[pl.ds(1, 56), pl.ds(0, 56), :])
    m = jnp.maximum(m, s11[pl.ds(1, 56), pl.ds(1, 56), :])
    o_ref[...] = m.reshape(56 * 56, 64).astype(jnp.bfloat16)


def _pool(y0q, stats0):
    s0, q0, g0, b0 = stats0
    body = functools.partial(_pool_body, count=float(64 * 112 * 112))

    def q_spec(ph):
        return pl.BlockSpec((1, 1, 56, 56, 64),
                            lambda i: (i, ph, 0, 0, 0))

    return pl.pallas_call(
        body,
        out_shape=jax.ShapeDtypeStruct((64 * 56 * 56, 64), jnp.bfloat16),
        grid_spec=pltpu.PrefetchScalarGridSpec(
            num_scalar_prefetch=0,
            grid=(64,),
            in_specs=[
                q_spec(0), q_spec(1), q_spec(2), q_spec(3),
                pl.BlockSpec((64, 1, 64), lambda i: (0, 0, 0)),
                pl.BlockSpec((64, 1, 64), lambda i: (0, 0, 0)),
                pl.BlockSpec((1, 64), lambda i: (0, 0)),
                pl.BlockSpec((1, 64), lambda i: (0, 0)),
            ],
            out_specs=pl.BlockSpec((56 * 56, 64), lambda i: (i, 0)),
            scratch_shapes=[
                pltpu.VMEM((56, 57, 64), jnp.float32),
                pltpu.VMEM((57, 56, 64), jnp.float32),
                pltpu.VMEM((57, 57, 64), jnp.float32),
            ]),
        compiler_params=pltpu.CompilerParams(
            dimension_semantics=("parallel",)),
    )(y0q, y0q, y0q, y0q, s0, q0, g0.reshape(1, 64).astype(jnp.float32),
      b0.reshape(1, 64).astype(jnp.float32))


def _head_body(x_ref, w_ref, b_ref, o_ref):
    xm = jnp.mean(x_ref[...].astype(jnp.float32), axis=1)
    o_ref[...] = (jnp.dot(xm, w_ref[...],
                          preferred_element_type=jnp.float32) + b_ref[...])


def _head(x4, fcw, fcb):
    x3 = x4.reshape(64, 49, 512)
    wT = jnp.transpose(fcw).astype(jnp.float32)
    b2 = fcb.reshape(1, 1000).astype(jnp.float32)
    return pl.pallas_call(
        _head_body,
        out_shape=jax.ShapeDtypeStruct((64, 1000), jnp.float32),
        grid_spec=pltpu.PrefetchScalarGridSpec(
            num_scalar_prefetch=0,
            grid=(1,),
            in_specs=[
                pl.BlockSpec((64, 49, 512), lambda i: (0, 0, 0)),
                pl.BlockSpec((512, 1000), lambda i: (0, 0)),
                pl.BlockSpec((1, 1000), lambda i: (0, 0)),
            ],
            out_specs=pl.BlockSpec((64, 1000), lambda i: (0, 0))),
        compiler_params=pltpu.CompilerParams(
            dimension_semantics=("arbitrary",)),
    )(x3, wT, b2)


def _w9(w):
    OC, IC, KH, KW = w.shape
    return jnp.transpose(w, (2, 3, 1, 0)).reshape(KH * KW, IC, OC).astype(
        jnp.bfloat16)


def _stage(x2, p0, p1, *, H, C, OC, stride, NB_in, NB_out):
    OH = H // stride
    count_o = float(64 * OH * OH)
    wd = p0.get('down_w')
    if stride == 2:
        ph = _phases(x2, H, C, pad=True)
        y1, s1, q1, yd, sd, qd = _conv_s2(
            ph, _w9(p0['conv1_w']), _w9(wd), NB=NB_in, H=H, C=C, OC=OC)
        id2 = yd
        statsd = (sd, qd, p0['down_bn_g'], p0['down_bn_b'])
    else:
        y1, s1, q1 = _conv(
            x2, _w9(p0['conv1_w']), NB=NB_in, H=H, W=H, C=C, OC=OC)
        id2, statsd = x2, None
    y2, s2, q2 = _conv(
        y1, _w9(p0['conv2_w']), NB=NB_out, H=OH, W=OH, C=OC, OC=OC,
        stats_in=(s1, q1, p0['bn1_g'], p0['bn1_b']))
    out0, y1b, s1b, q1b = _conv(
        y2, _w9(p1['conv1_w']), NB=NB_out, H=OH, W=OH, C=OC, OC=OC,
        stats_in=(s2, q2, p0['bn2_g'], p0['bn2_b']),
        residual=(id2, statsd))
    y2b, s2b, q2b = _conv(
        y1b, _w9(p1['conv2_w']), NB=NB_out, H=OH, W=OH, C=OC, OC=OC,
        stats_in=(s1b, q1b, p1['bn1_g'], p1['bn1_b']))
    return _residual(y2b, (s2b, q2b, p1['bn2_g'], p1['bn2_b']), out0, None,
                     rows=64 * OH * OH, C=OC, G=64 // NB_out,
                     count=count_o, count_d=count_o)


def kernel(x, c1w, b1g, b1b, l1_0_c1w, l1_0_b1g, l1_0_b1b, l1_0_c2w, l1_0_b2g, l1_0_b2b, l1_1_c1w, l1_1_b1g, l1_1_b1b, l1_1_c2w, l1_1_b2g, l1_1_b2b, l2_0_c1w, l2_0_b1g, l2_0_b1b, l2_0_c2w, l2_0_b2g, l2_0_b2b, l2_0_dw, l2_0_dbg, l2_0_dbb, l2_1_c1w, l2_1_b1g, l2_1_b1b, l2_1_c2w, l2_1_b2g, l2_1_b2b, l3_0_c1w, l3_0_b1g, l3_0_b1b, l3_0_c2w, l3_0_b2g, l3_0_b2b, l3_0_dw, l3_0_dbg, l3_0_dbb, l3_1_c1w, l3_1_b1g, l3_1_b1b, l3_1_c2w, l3_1_b2g, l3_1_b2b, l4_0_c1w, l4_0_b1g, l4_0_b1b, l4_0_c2w, l4_0_b2g, l4_0_b2b, l4_0_dw, l4_0_dbg, l4_0_dbb, l4_1_c1w, l4_1_b1g, l4_1_b1b, l4_1_c2w, l4_1_b2g, l4_1_b2b, fcw, fcb):
    xp = jnp.pad(x, ((0, 0), (0, 0), (3, 5), (3, 5)))
    xqs = xp.reshape(64, 3, 58, 2, 2, 58, 2, 2).transpose(
        0, 3, 6, 2, 5, 4, 7, 1).reshape(64, 4, 58, 58, 12).astype(
        jnp.bfloat16)
    w8 = jnp.pad(c1w, ((0, 0), (0, 0), (0, 1), (0, 1)))
    w16 = w8.reshape(64, 3, 4, 2, 4, 2).transpose(
        2, 4, 3, 5, 1, 0).reshape(16, 12, 64).astype(jnp.bfloat16)
    y0q, s0, q0 = _stem(xqs, w16)
    x1 = _pool(y0q, (s0, q0, b1g, b1b))

    p_l1_0 = {'conv1_w': l1_0_c1w, 'bn1_g': l1_0_b1g, 'bn1_b': l1_0_b1b,
              'conv2_w': l1_0_c2w, 'bn2_g': l1_0_b2g, 'bn2_b': l1_0_b2b}
    p_l1_1 = {'conv1_w': l1_1_c1w, 'bn1_g': l1_1_b1g, 'bn1_b': l1_1_b1b,
              'conv2_w': l1_1_c2w, 'bn2_g': l1_1_b2g, 'bn2_b': l1_1_b2b}
    p_l2_0 = {'conv1_w': l2_0_c1w, 'bn1_g': l2_0_b1g, 'bn1_b': l2_0_b1b,
              'conv2_w': l2_0_c2w, 'bn2_g': l2_0_b2g, 'bn2_b': l2_0_b2b,
              'down_w': l2_0_dw, 'down_bn_g': l2_0_dbg, 'down_bn_b': l2_0_dbb}
    p_l2_1 = {'conv1_w': l2_1_c1w, 'bn1_g': l2_1_b1g, 'bn1_b': l2_1_b1b,
              'conv2_w': l2_1_c2w, 'bn2_g': l2_1_b2g, 'bn2_b': l2_1_b2b}
    p_l3_0 = {'conv1_w': l3_0_c1w, 'bn1_g': l3_0_b1g, 'bn1_b': l3_0_b1b,
              'conv2_w': l3_0_c2w, 'bn2_g': l3_0_b2g, 'bn2_b': l3_0_b2b,
              'down_w': l3_0_dw, 'down_bn_g': l3_0_dbg, 'down_bn_b': l3_0_dbb}
    p_l3_1 = {'conv1_w': l3_1_c1w, 'bn1_g': l3_1_b1g, 'bn1_b': l3_1_b1b,
              'conv2_w': l3_1_c2w, 'bn2_g': l3_1_b2g, 'bn2_b': l3_1_b2b}
    p_l4_0 = {'conv1_w': l4_0_c1w, 'bn1_g': l4_0_b1g, 'bn1_b': l4_0_b1b,
              'conv2_w': l4_0_c2w, 'bn2_g': l4_0_b2g, 'bn2_b': l4_0_b2b,
              'down_w': l4_0_dw, 'down_bn_g': l4_0_dbg, 'down_bn_b': l4_0_dbb}
    p_l4_1 = {'conv1_w': l4_1_c1w, 'bn1_g': l4_1_b1g, 'bn1_b': l4_1_b1b,
              'conv2_w': l4_1_c2w, 'bn2_g': l4_1_b2g, 'bn2_b': l4_1_b2b}

    x = _stage(x1, p_l1_0, p_l1_1, H=56, C=64, OC=64, stride=1,
               NB_in=1, NB_out=1)
    x = _stage(x, p_l2_0, p_l2_1, H=56, C=64, OC=128, stride=2,
               NB_in=4, NB_out=4)
    x = _stage(x, p_l3_0, p_l3_1, H=28, C=128, OC=256, stride=2,
               NB_in=8, NB_out=8)
    x = _stage(x, p_l4_0, p_l4_1, H=14, C=256, OC=512, stride=2,
               NB_in=16, NB_out=16)

    return _head(x, fcw, fcb)


# speedup vs baseline: 5.8735x; 1.0016x over previous
import functools

import jax
import jax.numpy as jnp
from jax.experimental import pallas as pl
from jax.experimental.pallas import tpu as pltpu

_EPS = 1e-5


def _bn_coeffs(sum_ref, ssq_ref, g_ref, b_ref, count):
    s = jnp.sum(sum_ref[...], axis=0)
    q = jnp.sum(ssq_ref[...], axis=0)
    inv = 1.0 / count
    mean = s * inv
    var = jnp.maximum(q * inv - mean * mean, 0.0)
    scale = g_ref[...] * jax.lax.rsqrt(var + _EPS)
    shift = b_ref[...] - mean * scale
    return scale, shift


def _conv_body(*refs, NB, H, W, C, OC, mode, count_in):
    it = iter(refs)
    x_ref = next(it)
    if mode != 'plain':
        psum, pssq, g_ref, b_ref = next(it), next(it), next(it), next(it)
    if mode in ('res', 'res_bn'):
        id_ref = next(it)
    if mode == 'res_bn':
        dsum, dssq, dg_ref, db_ref = next(it), next(it), next(it), next(it)
    w_ref = next(it)
    ores_ref = next(it) if mode in ('res', 'res_bn') else None
    o_ref, os_ref, oq_ref = next(it), next(it), next(it)
    pr = next(it)

    M = NB * H * W
    xb = x_ref[...]
    if mode == 'plain':
        a = xb
    else:
        scale, shift = _bn_coeffs(psum, pssq, g_ref, b_ref, count_in)
        a = xb.astype(jnp.float32) * scale + shift
        if mode == 'res':
            a = a + id_ref[...].astype(jnp.float32)
        elif mode == 'res_bn':
            ds, dh = _bn_coeffs(dsum, dssq, dg_ref, db_ref, count_in)
            a = a + id_ref[...].astype(jnp.float32) * ds + dh
        a = jnp.maximum(a, 0.0).astype(jnp.bfloat16)
    if ores_ref is not None:
        ores_ref[...] = a

    pr[...] = jnp.zeros_like(pr)
    pr[:, pl.ds(1, H), pl.ds(1, W), :] = a.reshape(NB, H, W, C)

    acc = None
    for dy in range(3):
        for dx in range(3):
            sl = pr[:, pl.ds(dy, H), pl.ds(dx, W), :]
            at = sl.reshape(M, C)
            d = jnp.dot(at, w_ref[dy * 3 + dx],
                        preferred_element_type=jnp.float32)
            acc = d if acc is None else acc + d

    o_ref[...] = acc.astype(jnp.bfloat16)
    os_ref[...] = jnp.sum(acc, axis=0, keepdims=True)[None]
    oq_ref[...] = jnp.sum(acc * acc, axis=0, keepdims=True)[None]


def _conv_s2_body(p00, p01, p10, p11, w_ref, wd_ref, o_ref, os_ref, oq_ref,
                  od_ref, ods_ref, odq_ref, *, NB, OH, C):
    ph = (p00, p01, p10, p11)
    M = NB * OH * OH
    acc = None
    for dy in range(3):
        for dx in range(3):
            ref = ph[(dy % 2) * 2 + (dx % 2)]
            sl = ref[:, pl.ds(dy // 2, OH), pl.ds(dx // 2, OH), :]
            at = sl.reshape(M, C)
            d = jnp.dot(at, w_ref[dy * 3 + dx],
                        preferred_element_type=jnp.float32)
            acc = d if acc is None else acc + d
    o_ref[...] = acc.astype(jnp.bfloat16)
    os_ref[...] = jnp.sum(acc, axis=0, keepdims=True)[None]
    oq_ref[...] = jnp.sum(acc * acc, axis=0, keepdims=True)[None]

    ad = p11[:, pl.ds(0, OH), pl.ds(0, OH), :]
    accd = jnp.dot(ad.reshape(M, C), wd_ref[0],
                   preferred_element_type=jnp.float32)
    od_ref[...] = accd.astype(jnp.bfloat16)
    ods_ref[...] = jnp.sum(accd, axis=0, keepdims=True)[None]
    odq_ref[...] = jnp.sum(accd * accd, axis=0, keepdims=True)[None]


def _stat_specs(G, OC):
    return [
        pl.BlockSpec((1, 1, OC), lambda i: (i, 0, 0)),
        pl.BlockSpec((1, 1, OC), lambda i: (i, 0, 0)),
    ]


def _stat_shapes(G, OC):
    return [
        jax.ShapeDtypeStruct((G, 1, OC), jnp.float32),
        jax.ShapeDtypeStruct((G, 1, OC), jnp.float32),
    ]


def _conv(x2, w9, *, NB, H, W, C, OC, stats_in=None, residual=None):
    N = 64
    G = N // NB
    M = NB * H * W
    if stats_in is None:
        mode = 'plain'
    elif residual is None:
        mode = 'bn'
    else:
        mode = 'res' if residual[1] is None else 'res_bn'

    def vecf(v):
        return v.reshape(1, C).astype(jnp.float32)

    def stat_in_specs(gp):
        return [
            pl.BlockSpec((gp, 1, C), lambda i: (0, 0, 0)),
            pl.BlockSpec((gp, 1, C), lambda i: (0, 0, 0)),
            pl.BlockSpec((1, C), lambda i: (0, 0)),
            pl.BlockSpec((1, C), lambda i: (0, 0)),
        ]

    in_specs = [pl.BlockSpec((M, C), lambda i: (i, 0))]
    args = [x2]
    if mode != 'plain':
        s_in, q_in, g_in, b_in = stats_in
        in_specs += stat_in_specs(s_in.shape[0])
        args += [s_in, q_in, vecf(g_in), vecf(b_in)]
    if mode in ('res', 'res_bn'):
        id2, statsd = residual
        in_specs.append(pl.BlockSpec((M, C), lambda i: (i, 0)))
        args.append(id2)
        if statsd is not None:
            sd, qd, gd, bd = statsd
            in_specs += stat_in_specs(sd.shape[0])
            args += [sd, qd, vecf(gd), vecf(bd)]
    in_specs.append(pl.BlockSpec((9, C, OC), lambda i: (0, 0, 0)))
    args.append(w9)

    out_shape = [jax.ShapeDtypeStruct((N * H * W, OC), jnp.bfloat16)]
    out_specs = [pl.BlockSpec((M, OC), lambda i: (i, 0))]
    if mode in ('res', 'res_bn'):
        out_shape = [jax.ShapeDtypeStruct((N * H * W, C), jnp.bfloat16)
                     ] + out_shape
        out_specs = [pl.BlockSpec((M, C), lambda i: (i, 0))] + out_specs

    body = functools.partial(_conv_body, NB=NB, H=H, W=W, C=C, OC=OC,
                             mode=mode, count_in=float(N * H * W))

    return pl.pallas_call(
        body,
        out_shape=out_shape + _stat_shapes(G, OC),
        grid_spec=pltpu.PrefetchScalarGridSpec(
            num_scalar_prefetch=0,
            grid=(G,),
            in_specs=in_specs,
            out_specs=out_specs + _stat_specs(G, OC),
            scratch_shapes=[
                pltpu.VMEM((NB, H + 2, W + 2, C), jnp.bfloat16)]),
        compiler_params=pltpu.CompilerParams(
            dimension_semantics=("parallel",)),
    )(*args)


def _phases(x2, H, C, pad):
    x4 = x2.reshape(64, H, H, C)
    if pad:
        x4 = jnp.pad(x4, ((0, 0), (1, 1), (1, 1), (0, 0)))
    return [x4[:, r::2, s::2, :] for r in (0, 1) for s in (0, 1)]


def _conv_s2(ph, w9, wd, *, NB, H, C, OC):
    N = 64
    G = N // NB
    OH = H // 2
    PH = (H + 2) // 2
    M = NB * OH * OH
    OCD = wd.shape[2]

    ph_spec = pl.BlockSpec((NB, PH, PH, C), lambda i: (i, 0, 0, 0))
    body = functools.partial(_conv_s2_body, NB=NB, OH=OH, C=C)
    return pl.pallas_call(
        body,
        out_shape=[jax.ShapeDtypeStruct((N * OH * OH, OC), jnp.bfloat16)]
        + _stat_shapes(G, OC)
        + [jax.ShapeDtypeStruct((N * OH * OH, OCD), jnp.bfloat16)]
        + _stat_shapes(G, OCD),
        grid_spec=pltpu.PrefetchScalarGridSpec(
            num_scalar_prefetch=0,
            grid=(G,),
            in_specs=[ph_spec, ph_spec, ph_spec, ph_spec,
                      pl.BlockSpec((9, C, OC), lambda i: (0, 0, 0)),
                      pl.BlockSpec((1, C, OCD), lambda i: (0, 0, 0))],
            out_specs=[pl.BlockSpec((M, OC), lambda i: (i, 0))]
            + _stat_specs(G, OC)
            + [pl.BlockSpec((M, OCD), lambda i: (i, 0))]
            + _stat_specs(G, OCD)),
        compiler_params=pltpu.CompilerParams(
            dimension_semantics=("parallel",)),
    )(*ph, w9, wd)


def _residual_body(y_ref, ys_ref, yq_ref, yg_ref, yb_ref, r_ref, *rest,
                   count, count_d, has_dstats):
    if has_dstats:
        rs_ref, rq_ref, rg_ref, rb_ref, o_ref = rest
    else:
        (o_ref,) = rest
    scale, shift = _bn_coeffs(ys_ref, yq_ref, yg_ref, yb_ref, count)
    y = y_ref[...].astype(jnp.float32) * scale + shift
    if has_dstats:
        ds, dh = _bn_coeffs(rs_ref, rq_ref, rg_ref, rb_ref, count_d)
        r = r_ref[...].astype(jnp.float32) * ds + dh
    else:
        r = r_ref[...].astype(jnp.float32)
    o_ref[...] = jnp.maximum(y + r, 0.0).astype(jnp.bfloat16)


def _residual(y2, stats2, res2, statsd, *, rows, C, G, count, count_d):
    TR = rows // G
    s2, q2, g2, b2 = stats2
    gp = s2.shape[0]
    row_spec = pl.BlockSpec((TR, C), lambda i: (i, 0))
    st_spec = pl.BlockSpec((gp, 1, C), lambda i: (0, 0, 0))
    vec_spec = pl.BlockSpec((1, C), lambda i: (0, 0))
    in_specs = [row_spec, st_spec, st_spec, vec_spec, vec_spec, row_spec]
    args = [y2, s2, q2, g2.reshape(1, C).astype(jnp.float32),
            b2.reshape(1, C).astype(jnp.float32), res2]
    if statsd is not None:
        sd, qd, gd, bd = statsd
        gpd = sd.shape[0]
        std_spec = pl.BlockSpec((gpd, 1, C), lambda i: (0, 0, 0))
        in_specs += [std_spec, std_spec, vec_spec, vec_spec]
        args += [sd, qd, gd.reshape(1, C).astype(jnp.float32),
                 bd.reshape(1, C).astype(jnp.float32)]
    body = functools.partial(_residual_body, count=count, count_d=count_d,
                             has_dstats=statsd is not None)
    return pl.pallas_call(
        body,
        out_shape=jax.ShapeDtypeStruct((rows, C), jnp.bfloat16),
        grid_spec=pltpu.PrefetchScalarGridSpec(
            num_scalar_prefetch=0,
            grid=(G,),
            in_specs=in_specs,
            out_specs=row_spec),
        compiler_params=pltpu.CompilerParams(
            dimension_semantics=("parallel",)),
    )(*args)


def _stem_body(q00, q01, q10, q11, w_ref, o_ref, os_ref, oq_ref, acc_ref):
    qs = (q00, q01, q10, q11)
    ssum = None
    for r in (0, 1):
        for c in (0, 1):
            for a in range(4):
                for b in range(4):
                    u, v = r + a, c + b
                    ref = qs[(u % 2) * 2 + (v % 2)]
                    sl = ref[0, 0, pl.ds(u // 2, 56), pl.ds(v // 2, 56), :]
                    at = sl.reshape(56 * 56, 12)
                    d = jnp.dot(at, w_ref[a * 4 + b],
                                preferred_element_type=jnp.float32)
                    if a == 0 and b == 0:
                        acc_ref[...] = d
                    else:
                        acc_ref[...] += d
            acc = acc_ref[...]
            o_ref[0, r * 2 + c] = acc.reshape(56, 56, 64).astype(jnp.bfloat16)
            s1 = jnp.sum(acc, axis=0, keepdims=True)
            s2 = jnp.sum(acc * acc, axis=0, keepdims=True)
            ssum = (s1, s2) if ssum is None else (ssum[0] + s1, ssum[1] + s2)
    os_ref[...] = ssum[0][None]
    oq_ref[...] = ssum[1][None]


def _stem(xqs, w16):
    def q_spec(ph):
        return pl.BlockSpec((1, 1, 58, 58, 12),
                            lambda i: (i, ph, 0, 0, 0))
    return pl.pallas_call(
        _stem_body,
        out_shape=[
            jax.ShapeDtypeStruct((64, 4, 56, 56, 64), jnp.bfloat16),
            jax.ShapeDtypeStruct((64, 1, 64), jnp.float32),
            jax.ShapeDtypeStruct((64, 1, 64), jnp.float32),
        ],
        grid_spec=pltpu.PrefetchScalarGridSpec(
            num_scalar_prefetch=0,
            grid=(64,),
            in_specs=[q_spec(0), q_spec(1), q_spec(2), q_spec(3),
                      pl.BlockSpec((16, 12, 64), lambda i: (0, 0, 0))],
            out_specs=[
                pl.BlockSpec((1, 4, 56, 56, 64), lambda i: (i, 0, 0, 0, 0)),
                pl.BlockSpec((1, 1, 64), lambda i: (i, 0, 0)),
                pl.BlockSpec((1, 1, 64), lambda i: (i, 0, 0)),
            ],
            scratch_shapes=[pltpu.VMEM((56 * 56, 64), jnp.float32)]),
        compiler_params=pltpu.CompilerParams(
            dimension_semantics=("parallel",)),
    )(xqs, xqs, xqs, xqs, w16)


def _pool_body(q00, q01, q10, q11, psum, pssq, g_ref, b_ref, o_ref,
               s01, s10, s11, *, count):
    scale, shift = _bn_coeffs(psum, pssq, g_ref, b_ref, count)

    def bn(qref):
        v = qref[0, 0].astype(jnp.float32)
        return jnp.maximum(v * scale + shift, 0.0)

    s01[...] = jnp.full_like(s01, -jnp.inf)
    s01[:, pl.ds(1, 56), :] = bn(q01)
    s10[...] = jnp.full_like(s10, -jnp.inf)
    s10[pl.ds(1, 56), :, :] = bn(q10)
    s11[...] = jnp.full_like(s11, -jnp.inf)
    s11[pl.ds(1, 56), pl.ds(1, 56), :] = bn(q11)

    m = bn(q00)
    m = jnp.maximum(m, s01[:, pl.ds(0, 56), :])
    m = jnp.maximum(m, s01[:, pl.ds(1, 56), :])
    m = jnp.maximum(m, s10[pl.ds(0, 56), :, :])
    m = jnp.maximum(m, s10[pl.ds(1, 56), :, :])
    m = jnp.maximum(m, s11[pl.ds(0, 56), pl.ds(0, 56), :])
    m = jnp.maximum(m, s11[pl.ds(0, 56), pl.ds(1, 56), :])
    m = jnp.maximum(m, s11[pl.ds(1, 56), pl.ds(0, 56), :])
    m = jnp.maximum(m, s11[pl.ds(1, 56), pl.ds(1, 56), :])
    o_ref[...] = m.reshape(56 * 56, 64).astype(jnp.bfloat16)


def _pool(y0q, stats0):
    s0, q0, g0, b0 = stats0
    body = functools.partial(_pool_body, count=float(64 * 112 * 112))

    def q_spec(ph):
        return pl.BlockSpec((1, 1, 56, 56, 64),
                            lambda i: (i, ph, 0, 0, 0))

    return pl.pallas_call(
        body,
        out_shape=jax.ShapeDtypeStruct((64 * 56 * 56, 64), jnp.bfloat16),
        grid_spec=pltpu.PrefetchScalarGridSpec(
            num_scalar_prefetch=0,
            grid=(64,),
            in_specs=[
                q_spec(0), q_spec(1), q_spec(2), q_spec(3),
                pl.BlockSpec((64, 1, 64), lambda i: (0, 0, 0)),
                pl.BlockSpec((64, 1, 64), lambda i: (0, 0, 0)),
                pl.BlockSpec((1, 64), lambda i: (0, 0)),
                pl.BlockSpec((1, 64), lambda i: (0, 0)),
            ],
            out_specs=pl.BlockSpec((56 * 56, 64), lambda i: (i, 0)),
            scratch_shapes=[
                pltpu.VMEM((56, 57, 64), jnp.float32),
                pltpu.VMEM((57, 56, 64), jnp.float32),
                pltpu.VMEM((57, 57, 64), jnp.float32),
            ]),
        compiler_params=pltpu.CompilerParams(
            dimension_semantics=("parallel",)),
    )(y0q, y0q, y0q, y0q, s0, q0, g0.reshape(1, 64).astype(jnp.float32),
      b0.reshape(1, 64).astype(jnp.float32))


def _head_body(x_ref, w_ref, b_ref, o_ref):
    xm = jnp.mean(x_ref[...].astype(jnp.float32), axis=1)
    o_ref[...] = (jnp.dot(xm, w_ref[...],
                          preferred_element_type=jnp.float32) + b_ref[...])


def _head(x4, fcw, fcb):
    x3 = x4.reshape(64, 49, 512)
    wT = jnp.transpose(fcw).astype(jnp.float32)
    b2 = fcb.reshape(1, 1000).astype(jnp.float32)
    return pl.pallas_call(
        _head_body,
        out_shape=jax.ShapeDtypeStruct((64, 1000), jnp.float32),
        grid_spec=pltpu.PrefetchScalarGridSpec(
            num_scalar_prefetch=0,
            grid=(1,),
            in_specs=[
                pl.BlockSpec((64, 49, 512), lambda i: (0, 0, 0)),
                pl.BlockSpec((512, 1000), lambda i: (0, 0)),
                pl.BlockSpec((1, 1000), lambda i: (0, 0)),
            ],
            out_specs=pl.BlockSpec((64, 1000), lambda i: (0, 0))),
        compiler_params=pltpu.CompilerParams(
            dimension_semantics=("arbitrary",)),
    )(x3, wT, b2)


def _w9(w):
    OC, IC, KH, KW = w.shape
    return jnp.transpose(w, (2, 3, 1, 0)).reshape(KH * KW, IC, OC).astype(
        jnp.bfloat16)


def _stage(x2, p0, p1, *, H, C, OC, stride, NB_in, NB_out):
    OH = H // stride
    count_o = float(64 * OH * OH)
    wd = p0.get('down_w')
    if stride == 2:
        ph = _phases(x2, H, C, pad=True)
        y1, s1, q1, yd, sd, qd = _conv_s2(
            ph, _w9(p0['conv1_w']), _w9(wd), NB=NB_in, H=H, C=C, OC=OC)
        id2 = yd
        statsd = (sd, qd, p0['down_bn_g'], p0['down_bn_b'])
    else:
        y1, s1, q1 = _conv(
            x2, _w9(p0['conv1_w']), NB=NB_in, H=H, W=H, C=C, OC=OC)
        id2, statsd = x2, None
    y2, s2, q2 = _conv(
        y1, _w9(p0['conv2_w']), NB=NB_out, H=OH, W=OH, C=OC, OC=OC,
        stats_in=(s1, q1, p0['bn1_g'], p0['bn1_b']))
    out0, y1b, s1b, q1b = _conv(
        y2, _w9(p1['conv1_w']), NB=NB_out, H=OH, W=OH, C=OC, OC=OC,
        stats_in=(s2, q2, p0['bn2_g'], p0['bn2_b']),
        residual=(id2, statsd))
    y2b, s2b, q2b = _conv(
        y1b, _w9(p1['conv2_w']), NB=NB_out, H=OH, W=OH, C=OC, OC=OC,
        stats_in=(s1b, q1b, p1['bn1_g'], p1['bn1_b']))
    return _residual(y2b, (s2b, q2b, p1['bn2_g'], p1['bn2_b']), out0, None,
                     rows=64 * OH * OH, C=OC, G=64 // NB_out,
                     count=count_o, count_d=count_o)


def kernel(x, c1w, b1g, b1b, l1_0_c1w, l1_0_b1g, l1_0_b1b, l1_0_c2w, l1_0_b2g, l1_0_b2b, l1_1_c1w, l1_1_b1g, l1_1_b1b, l1_1_c2w, l1_1_b2g, l1_1_b2b, l2_0_c1w, l2_0_b1g, l2_0_b1b, l2_0_c2w, l2_0_b2g, l2_0_b2b, l2_0_dw, l2_0_dbg, l2_0_dbb, l2_1_c1w, l2_1_b1g, l2_1_b1b, l2_1_c2w, l2_1_b2g, l2_1_b2b, l3_0_c1w, l3_0_b1g, l3_0_b1b, l3_0_c2w, l3_0_b2g, l3_0_b2b, l3_0_dw, l3_0_dbg, l3_0_dbb, l3_1_c1w, l3_1_b1g, l3_1_b1b, l3_1_c2w, l3_1_b2g, l3_1_b2b, l4_0_c1w, l4_0_b1g, l4_0_b1b, l4_0_c2w, l4_0_b2g, l4_0_b2b, l4_0_dw, l4_0_dbg, l4_0_dbb, l4_1_c1w, l4_1_b1g, l4_1_b1b, l4_1_c2w, l4_1_b2g, l4_1_b2b, fcw, fcb):
    xp = jnp.pad(x, ((0, 0), (0, 0), (3, 5), (3, 5)))
    xqs = xp.reshape(64, 3, 58, 2, 2, 58, 2, 2).transpose(
        0, 3, 6, 2, 5, 4, 7, 1).reshape(64, 4, 58, 58, 12).astype(
        jnp.bfloat16)
    w8 = jnp.pad(c1w, ((0, 0), (0, 0), (0, 1), (0, 1)))
    w16 = w8.reshape(64, 3, 4, 2, 4, 2).transpose(
        2, 4, 3, 5, 1, 0).reshape(16, 12, 64).astype(jnp.bfloat16)
    y0q, s0, q0 = _stem(xqs, w16)
    x1 = _pool(y0q, (s0, q0, b1g, b1b))

    p_l1_0 = {'conv1_w': l1_0_c1w, 'bn1_g': l1_0_b1g, 'bn1_b': l1_0_b1b,
              'conv2_w': l1_0_c2w, 'bn2_g': l1_0_b2g, 'bn2_b': l1_0_b2b}
    p_l1_1 = {'conv1_w': l1_1_c1w, 'bn1_g': l1_1_b1g, 'bn1_b': l1_1_b1b,
              'conv2_w': l1_1_c2w, 'bn2_g': l1_1_b2g, 'bn2_b': l1_1_b2b}
    p_l2_0 = {'conv1_w': l2_0_c1w, 'bn1_g': l2_0_b1g, 'bn1_b': l2_0_b1b,
              'conv2_w': l2_0_c2w, 'bn2_g': l2_0_b2g, 'bn2_b': l2_0_b2b,
              'down_w': l2_0_dw, 'down_bn_g': l2_0_dbg, 'down_bn_b': l2_0_dbb}
    p_l2_1 = {'conv1_w': l2_1_c1w, 'bn1_g': l2_1_b1g, 'bn1_b': l2_1_b1b,
              'conv2_w': l2_1_c2w, 'bn2_g': l2_1_b2g, 'bn2_b': l2_1_b2b}
    p_l3_0 = {'conv1_w': l3_0_c1w, 'bn1_g': l3_0_b1g, 'bn1_b': l3_0_b1b,
              'conv2_w': l3_0_c2w, 'bn2_g': l3_0_b2g, 'bn2_b': l3_0_b2b,
              'down_w': l3_0_dw, 'down_bn_g': l3_0_dbg, 'down_bn_b': l3_0_dbb}
    p_l3_1 = {'conv1_w': l3_1_c1w, 'bn1_g': l3_1_b1g, 'bn1_b': l3_1_b1b,
              'conv2_w': l3_1_c2w, 'bn2_g': l3_1_b2g, 'bn2_b': l3_1_b2b}
    p_l4_0 = {'conv1_w': l4_0_c1w, 'bn1_g': l4_0_b1g, 'bn1_b': l4_0_b1b,
              'conv2_w': l4_0_c2w, 'bn2_g': l4_0_b2g, 'bn2_b': l4_0_b2b,
              'down_w': l4_0_dw, 'down_bn_g': l4_0_dbg, 'down_bn_b': l4_0_dbb}
    p_l4_1 = {'conv1_w': l4_1_c1w, 'bn1_g': l4_1_b1g, 'bn1_b': l4_1_b1b,
              'conv2_w': l4_1_c2w, 'bn2_g': l4_1_b2g, 'bn2_b': l4_1_b2b}

    x = _stage(x1, p_l1_0, p_l1_1, H=56, C=64, OC=64, stride=1,
               NB_in=2, NB_out=2)
    x = _stage(x, p_l2_0, p_l2_1, H=56, C=64, OC=128, stride=2,
               NB_in=8, NB_out=8)
    x = _stage(x, p_l3_0, p_l3_1, H=28, C=128, OC=256, stride=2,
               NB_in=8, NB_out=8)
    x = _stage(x, p_l4_0, p_l4_1, H=14, C=256, OC=512, stride=2,
               NB_in=16, NB_out=16)

    return _head(x, fcw, fcb)
